# baseline (speedup 1.0000x reference)
import functools

import jax
import jax.numpy as jnp
from jax import lax
from jax.experimental import pallas as pl
from jax.experimental.pallas import tpu as pltpu
from jax.experimental.pallas import tpu_sc as plsc

N = 10000
E = 320000
D = 128
L = 4

NC = 2
NS = 16
NW = NC * NS
CH = 128
CHUNKS = E // CH
E2 = E // 2
HCHUNKS = E2 // CH
ROWS_PER_TILE = 632
NP = ROWS_PER_TILE * NS

@functools.lru_cache(maxsize=None)
def _sc_mesh():
    return plsc.VectorSubcoreMesh(
        core_axis_name="c", subcore_axis_name="s",
        num_cores=NC, num_subcores=NS)


def _silu(v):
    return v * (1.0 / (1.0 + jnp.exp(-v)))


def _rne_bf16_bits(x):
    rb = jax.lax.bitcast_convert_type(x, jnp.uint32)
    return (rb + jnp.uint32(0x7FFF) + ((rb >> 16) & jnp.uint32(1))) >> 16


def _pack_bf16_pair(lo, hi):
    w = _rne_bf16_bits(lo) | (_rne_bf16_bits(hi) << 16)
    return jax.lax.bitcast_convert_type(w, jnp.int32)


def _unpack_bf16_pair(w):
    u = jax.lax.bitcast_convert_type(w, jnp.uint32)
    lo = jax.lax.bitcast_convert_type(u << 16, jnp.float32)
    hi = jax.lax.bitcast_convert_type(u & jnp.uint32(0xFFFF0000), jnp.float32)
    return lo, hi


MAXC = 40


def _pad_worker_idx(idx):
    idx2d = idx.reshape(HCHUNKS, CH)
    per = HCHUNKS // NW
    rem = HCHUNKS % NW
    slabs = []
    for w in range(NW):
        b = w * per + min(w, rem)
        cnt = per + (1 if w < rem else 0)
        slabs.append(jnp.pad(idx2d[b:b + cnt], ((0, MAXC - cnt), (0, 0))))
    return jnp.stack(slabs)


def _worker_split(wid):
    per = HCHUNKS // NW
    rem = HCHUNKS % NW
    base = wid * per + jnp.minimum(wid, rem)
    cnt = per + jnp.where(wid < rem, 1, 0)
    return base, cnt


NBUF = 3


def _gather_add_body(p_hbm, q_hbm, row_hbm, col_hbm, z_hbm,
                     ridx, cidx, pbuf0, qbuf0, pbuf1, qbuf1,
                     isem, gsem0, gsem1, wsem0, wsem1):
    c = lax.axis_index("c")
    s = lax.axis_index("s")
    wid = s * NC + c
    base, cnt = _worker_split(wid)

    pltpu.async_copy(row_hbm.at[wid], ridx, isem)
    pltpu.async_copy(col_hbm.at[wid], cidx, isem)
    pltpu.make_async_copy(row_hbm.at[wid], ridx, isem).wait()
    pltpu.make_async_copy(col_hbm.at[wid], cidx, isem).wait()

    pbufs = (pbuf0, pbuf1)
    qbufs = (qbuf0, qbuf1)
    gsems = (gsem0, gsem1)
    wsems = (wsem0, wsem1)

    def _issue(j, b):
        pltpu.async_copy(p_hbm.at[ridx.at[j]], pbufs[b], gsems[b])
        pltpu.async_copy(q_hbm.at[cidx.at[j]], qbufs[b], gsems[b])

    def _process(j, b):
        pltpu.make_async_copy(p_hbm.at[ridx.at[j]], pbufs[b], gsems[b]).wait()
        pltpu.make_async_copy(q_hbm.at[cidx.at[j]], qbufs[b], gsems[b]).wait()
        pb, qb = pbufs[b], qbufs[b]

        def add_rows(i, carry):
            r = i * 4
            for rr in range(4):
                for cc in range(D // 16):
                    sl = pl.ds(cc * 16, 16)
                    pb[r + rr, sl] = pb[r + rr, sl] + qb[r + rr, sl]
            return carry

        lax.fori_loop(0, CH // 4, add_rows, 0)
        pltpu.async_copy(pb, z_hbm.at[pl.ds((base + j) * CH, CH)], wsems[b])

    def step(i, carry):
        for b in range(2):
            j = i * 2 + b

            @pl.when(j < cnt)
            def _():
                @pl.when(j >= 2)
                def _():
                    pltpu.make_async_copy(
                        pbufs[b], z_hbm.at[pl.ds(0, CH)], wsems[b]).wait()

                _issue(j, b)

            @pl.when((j >= 1) & (j <= cnt))
            def _():
                _process(j - 1, 1 - b)
        return carry

    lax.fori_loop(0, (cnt + 2) // 2, step, 0)
    pltpu.make_async_copy(pbuf0, z_hbm.at[pl.ds(0, CH)], wsem0).wait()
    pltpu.make_async_copy(pbuf1, z_hbm.at[pl.ds(0, CH)], wsem1).wait()


@functools.lru_cache(maxsize=None)
def _gather_add_kernel():
    return pl.kernel(
        _gather_add_body,
        out_type=jax.ShapeDtypeStruct((E2, D), jnp.float32),
        mesh=_sc_mesh(),
        scratch_types=[
            pltpu.VMEM((MAXC, CH), jnp.int32),
            pltpu.VMEM((MAXC, CH), jnp.int32),
            pltpu.VMEM((CH, D), jnp.float32),
            pltpu.VMEM((CH, D), jnp.float32),
            pltpu.VMEM((CH, D), jnp.float32),
            pltpu.VMEM((CH, D), jnp.float32),
            pltpu.SemaphoreType.DMA,
            pltpu.SemaphoreType.DMA,
            pltpu.SemaphoreType.DMA,
            pltpu.SemaphoreType.DMA,
            pltpu.SemaphoreType.DMA,
        ],
    )


def _gather_add(p, q, rowp, colp):
    return _gather_add_kernel()(p, q, rowp, colp)


def _scatter_add_body(m_hbm, row_hbm, zeros_hbm, agg_hbm,
                      ridx, mbuf0, mbuf1, accum, lsem0, lsem1):
    c = lax.axis_index("c")
    s = lax.axis_index("s")
    wid = s * NC + c
    base, cnt = _worker_split(wid)
    pltpu.sync_copy(zeros_hbm.at[pl.ds(s * ROWS_PER_TILE, ROWS_PER_TILE)],
                    accum.at[pl.ds(s * ROWS_PER_TILE, ROWS_PER_TILE)])

    pltpu.sync_copy(row_hbm.at[wid], ridx)
    plsc.subcore_barrier()

    mbufs = (mbuf0, mbuf1)
    lsems = (lsem0, lsem1)

    def step(i, carry):
        for b in range(2):
            j = i * 2 + b

            @pl.when(j < cnt)
            def _():
                pltpu.async_copy(m_hbm.at[pl.ds((base + j) * CH, CH)],
                                 mbufs[b], lsems[b])

            @pl.when((j >= 1) & (j <= cnt))
            def _():
                pltpu.make_async_copy(
                    m_hbm.at[pl.ds(base * CH, CH)],
                    mbufs[1 - b], lsems[1 - b]).wait()
                pltpu.sync_copy(mbufs[1 - b], accum.at[ridx.at[j - 1]],
                                add=True)
        return carry

    lax.fori_loop(0, (cnt + 2) // 2, step, 0)
    plsc.subcore_barrier()
    r0 = s * ROWS_PER_TILE
    pltpu.sync_copy(accum.at[pl.ds(r0, ROWS_PER_TILE)],
                    agg_hbm.at[pl.ds(c * NP + r0, ROWS_PER_TILE)])


@functools.lru_cache(maxsize=None)
def _scatter_add_kernel():
    return pl.kernel(
        _scatter_add_body,
        out_type=jax.ShapeDtypeStruct((NC * NP, D), jnp.float32),
        mesh=_sc_mesh(),
        scratch_types=[
            pltpu.VMEM((MAXC, CH), jnp.int32),
            pltpu.VMEM((CH, D), jnp.float32),
            pltpu.VMEM((CH, D), jnp.float32),
            pltpu.VMEM_SHARED((NP, D), jnp.float32),
            pltpu.SemaphoreType.DMA,
            pltpu.SemaphoreType.DMA,
        ],
    )


def _scatter_add(m, rowp, zeros):
    return _scatter_add_kernel()(m, rowp, zeros)


BE = 4000
BN = 2000


def _edge_mlp_body(z_ref, ea_ref, ce_ref, co_ref, be_ref, bo_ref,
                   w2e_ref, w2o_ref, be2_ref, m_ref):
    z = z_ref[...]
    zlo = z[:, :D // 2]
    zhi = z[:, D // 2:]
    ea = ea_ref[...]
    ze = zlo + ea[:, 0:1] * ce_ref[0:1, :] + ea[:, 1:2] * ce_ref[1:2, :] + be_ref[...]
    zo = zhi + ea[:, 0:1] * co_ref[0:1, :] + ea[:, 1:2] * co_ref[1:2, :] + bo_ref[...]
    ae = _silu(ze)
    ao = _silu(zo)
    m = (jnp.dot(ae, w2e_ref[...], preferred_element_type=jnp.float32)
         + jnp.dot(ao, w2o_ref[...], preferred_element_type=jnp.float32)
         + be2_ref[...])
    m_ref[...] = _silu(m)


def _edge_mlp(z, ea, ce, co, be, bo, w2e, w2o, be2):
    grid = (E2 // BE,)
    H = D // 2
    return pl.pallas_call(
        _edge_mlp_body,
        grid=grid,
        in_specs=[
            pl.BlockSpec((BE, D), lambda i: (i, 0)),
            pl.BlockSpec((BE, 2), lambda i: (i, 0)),
            pl.BlockSpec((2, H), lambda i: (0, 0)),
            pl.BlockSpec((2, H), lambda i: (0, 0)),
            pl.BlockSpec((1, H), lambda i: (0, 0)),
            pl.BlockSpec((1, H), lambda i: (0, 0)),
            pl.BlockSpec((H, D), lambda i: (0, 0)),
            pl.BlockSpec((H, D), lambda i: (0, 0)),
            pl.BlockSpec((1, D), lambda i: (0, 0)),
        ],
        out_specs=pl.BlockSpec((BE, D), lambda i: (i, 0)),
        out_shape=jax.ShapeDtypeStruct((E2, D), jnp.float32),
    )(z, ea, ce, co, be, bo, w2e, w2o, be2)


def _node_body(h_ref, a0_ref, a1_ref, a2_ref, a3_ref, w1a_ref, w1b_ref,
               b1_ref, w2_ref, b2_ref, wpq_ref, h_out, p_out, q_out):
    h = h_ref[...]
    agg = (a0_ref[...] + a1_ref[...]) + (a2_ref[...] + a3_ref[...])
    u = (jnp.dot(h, w1a_ref[...], preferred_element_type=jnp.float32)
         + jnp.dot(agg, w1b_ref[...], preferred_element_type=jnp.float32)
         + b1_ref[...])
    u = _silu(u)
    hn = h + jnp.dot(u, w2_ref[...], preferred_element_type=jnp.float32) + b2_ref[...]
    h_out[...] = hn
    pq = jnp.dot(hn, wpq_ref[...], preferred_element_type=jnp.float32)
    p_out[...] = pq[:, :D]
    q_out[...] = pq[:, D:]


def _node_update(h, a0, a1, a2, a3, w1a, w1b, b1, w2, b2, wpq):
    grid = (N // BN,)
    full = lambda i: (0, 0)
    return pl.pallas_call(
        _node_body,
        grid=grid,
        in_specs=[
            pl.BlockSpec((BN, D), lambda i: (i, 0)),
            pl.BlockSpec((BN, D), lambda i: (i, 0)),
            pl.BlockSpec((BN, D), lambda i: (i, 0)),
            pl.BlockSpec((BN, D), lambda i: (i, 0)),
            pl.BlockSpec((BN, D), lambda i: (i, 0)),
            pl.BlockSpec((D, D), full),
            pl.BlockSpec((D, D), full),
            pl.BlockSpec((1, D), full),
            pl.BlockSpec((D, D), full),
            pl.BlockSpec((1, D), full),
            pl.BlockSpec((D, 2 * D), full),
        ],
        out_specs=[
            pl.BlockSpec((BN, D), lambda i: (i, 0)),
            pl.BlockSpec((BN, D), lambda i: (i, 0)),
            pl.BlockSpec((BN, D), lambda i: (i, 0)),
        ],
        out_shape=[
            jax.ShapeDtypeStruct((N, D), jnp.float32),
            jax.ShapeDtypeStruct((N, D), jnp.float32),
            jax.ShapeDtypeStruct((N, D), jnp.float32),
        ],
    )(h, a0, a1, a2, a3, w1a, w1b, b1, w2, b2, wpq)


def _embed_body(loc_ref, vel_ref, wl_ref, wv_ref, b_ref, wpq_ref,
                h_out, p_out, q_out):
    loc = loc_ref[...]
    vel = vel_ref[...]
    h = b_ref[...] + jnp.zeros((loc.shape[0], D), jnp.float32)
    for j in range(3):
        h = h + loc[:, j:j + 1] * wl_ref[j:j + 1, :]
        h = h + vel[:, j:j + 1] * wv_ref[j:j + 1, :]
    h_out[...] = h
    pq = jnp.dot(h, wpq_ref[...], preferred_element_type=jnp.float32)
    p_out[...] = pq[:, :D]
    q_out[...] = pq[:, D:]


def _embed(loc, vel, wl, wv, b, wpq):
    grid = (N // BN,)
    full = lambda i: (0, 0)
    return pl.pallas_call(
        _embed_body,
        grid=grid,
        in_specs=[
            pl.BlockSpec((BN, 3), lambda i: (i, 0)),
            pl.BlockSpec((BN, 3), lambda i: (i, 0)),
            pl.BlockSpec((3, D), full),
            pl.BlockSpec((3, D), full),
            pl.BlockSpec((1, D), full),
            pl.BlockSpec((D, 2 * D), full),
        ],
        out_specs=[
            pl.BlockSpec((BN, D), lambda i: (i, 0)),
            pl.BlockSpec((BN, D), lambda i: (i, 0)),
            pl.BlockSpec((BN, D), lambda i: (i, 0)),
        ],
        out_shape=[
            jax.ShapeDtypeStruct((N, D), jnp.float32),
            jax.ShapeDtypeStruct((N, D), jnp.float32),
            jax.ShapeDtypeStruct((N, D), jnp.float32),
        ],
    )(loc, vel, wl, wv, b, wpq)


def _node_decode_body(h_ref, a0_ref, a1_ref, a2_ref, a3_ref, w1a_ref,
                      w1b_ref, b1_ref, w2_ref, b2_ref, wd1_ref, bd1_ref,
                      wd2_ref, bd2_ref, o_ref):
    h = h_ref[...]
    agg = (a0_ref[...] + a1_ref[...]) + (a2_ref[...] + a3_ref[...])
    u = (jnp.dot(h, w1a_ref[...], preferred_element_type=jnp.float32)
         + jnp.dot(agg, w1b_ref[...], preferred_element_type=jnp.float32)
         + b1_ref[...])
    u = _silu(u)
    hn = h + jnp.dot(u, w2_ref[...], preferred_element_type=jnp.float32) + b2_ref[...]
    d = _silu(jnp.dot(hn, wd1_ref[...], preferred_element_type=jnp.float32)
              + bd1_ref[...])
    o_ref[...] = (jnp.dot(d, wd2_ref[...], preferred_element_type=jnp.float32)
                  + bd2_ref[...])


def _node_decode(h, a0, a1, a2, a3, w1a, w1b, b1, w2, b2, wd1, bd1, wd2, bd2):
    grid = (N // BN,)
    full = lambda i: (0, 0)
    blk = lambda i: (i, 0)
    return pl.pallas_call(
        _node_decode_body,
        grid=grid,
        in_specs=[
            pl.BlockSpec((BN, D), blk),
            pl.BlockSpec((BN, D), blk),
            pl.BlockSpec((BN, D), blk),
            pl.BlockSpec((BN, D), blk),
            pl.BlockSpec((BN, D), blk),
            pl.BlockSpec((D, D), full),
            pl.BlockSpec((D, D), full),
            pl.BlockSpec((1, D), full),
            pl.BlockSpec((D, D), full),
            pl.BlockSpec((1, D), full),
            pl.BlockSpec((D, D), full),
            pl.BlockSpec((1, D), full),
            pl.BlockSpec((D, 3), full),
            pl.BlockSpec((1, 3), full),
        ],
        out_specs=pl.BlockSpec((BN, 3), blk),
        out_shape=jax.ShapeDtypeStruct((N, 3), jnp.float32),
    )(h, a0, a1, a2, a3, w1a, w1b, b1, w2, b2, wd1, bd1, wd2, bd2)


def _decode_body(h_ref, w1_ref, b1_ref, w2_ref, b2_ref, o_ref):
    h = h_ref[...]
    d = _silu(jnp.dot(h, w1_ref[...], preferred_element_type=jnp.float32)
              + b1_ref[...])
    o_ref[...] = (jnp.dot(d, w2_ref[...], preferred_element_type=jnp.float32)
                  + b2_ref[...])


def _decode(h, w1, b1, w2, b2):
    grid = (N // BN,)
    full = lambda i: (0, 0)
    return pl.pallas_call(
        _decode_body,
        grid=grid,
        in_specs=[
            pl.BlockSpec((BN, D), lambda i: (i, 0)),
            pl.BlockSpec((D, D), full),
            pl.BlockSpec((1, D), full),
            pl.BlockSpec((D, 3), full),
            pl.BlockSpec((1, 3), full),
        ],
        out_specs=pl.BlockSpec((BN, 3), lambda i: (i, 0)),
        out_shape=jax.ShapeDtypeStruct((N, 3), jnp.float32),
    )(h, w1, b1, w2, b2)


def kernel(nodes, loc, edges, vel, edge_attr, _, W_emb, b_emb, We1, be1,
           We2, be2, Wn1, bn1, Wn2, bn2, Wd1, bd1, Wd2, bd2):
    row = edges[0]
    col = edges[1]
    rowp = [_pad_worker_idx(row[:E2]), _pad_worker_idx(row[E2:])]
    colp = [_pad_worker_idx(col[:E2]), _pad_worker_idx(col[E2:])]
    ea = [edge_attr[:E2], edge_attr[E2:]]
    zeros = jnp.zeros((NP, D), jnp.float32)

    def _wpq(i):
        A = We1[i, :D, :]
        B = We1[i, D:2 * D, :]
        return jnp.concatenate([A[:, 0::2], A[:, 1::2],
                                B[:, 0::2], B[:, 1::2]], axis=1)

    wpq = [_wpq(i) for i in range(L)]

    h, p, q = _embed(loc, vel, W_emb[:3], W_emb[3:], b_emb.reshape(1, D),
                     wpq[0])
    for i in range(L):
        c2 = We1[i, 2 * D:, :]
        ce = c2[:, 0::2]
        co = c2[:, 1::2]
        b1e = be1[i, 0::2].reshape(1, D // 2)
        b1o = be1[i, 1::2].reshape(1, D // 2)
        w2e = We2[i, 0::2, :]
        w2o = We2[i, 1::2, :]
        b2 = be2[i].reshape(1, D)
        z0 = _gather_add(p, q, rowp[0], colp[0])
        m0 = _edge_mlp(z0, ea[0], ce, co, b1e, b1o, w2e, w2o, b2)
        z1 = _gather_add(p, q, rowp[1], colp[1])
        agg0 = _scatter_add(m0, rowp[0], zeros)
        m1 = _edge_mlp(z1, ea[1], ce, co, b1e, b1o, w2e, w2o, b2)
        agg1 = _scatter_add(m1, rowp[1], zeros)
        if i < L - 1:
            h, p, q = _node_update(h, agg0[:N], agg0[NP:NP + N],
                                   agg1[:N], agg1[NP:NP + N],
                                   Wn1[i, :D, :], Wn1[i, D:, :],
                                   bn1[i].reshape(1, D), Wn2[i],
                                   bn2[i].reshape(1, D), wpq[i + 1])
        else:
            return _node_decode(h, agg0[:N], agg0[NP:NP + N],
                                agg1[:N], agg1[NP:NP + N],
                                Wn1[i, :D, :], Wn1[i, D:, :],
                                bn1[i].reshape(1, D), Wn2[i],
                                bn2[i].reshape(1, D),
                                Wd1, bd1.reshape(1, D), Wd2,
                                bd2.reshape(1, 3))


# speedup vs baseline: 1.0016x; 1.0016x over previous
import functools

import jax
import jax.numpy as jnp
from jax import lax
from jax.experimental import pallas as pl
from jax.experimental.pallas import tpu as pltpu
from jax.experimental.pallas import tpu_sc as plsc

N = 10000
E = 320000
D = 128
L = 4

NC = 2
NS = 16
NW = NC * NS
CH = 128
CHUNKS = E // CH
E2 = E // 2
HCHUNKS = E2 // CH
ROWS_PER_TILE = 632
NP = ROWS_PER_TILE * NS

@functools.lru_cache(maxsize=None)
def _sc_mesh():
    return plsc.VectorSubcoreMesh(
        core_axis_name="c", subcore_axis_name="s",
        num_cores=NC, num_subcores=NS)


def _silu(v):
    return v * (1.0 / (1.0 + jnp.exp(-v)))


def _rne_bf16_bits(x):
    rb = jax.lax.bitcast_convert_type(x, jnp.uint32)
    return (rb + jnp.uint32(0x7FFF) + ((rb >> 16) & jnp.uint32(1))) >> 16


def _pack_bf16_pair(lo, hi):
    w = _rne_bf16_bits(lo) | (_rne_bf16_bits(hi) << 16)
    return jax.lax.bitcast_convert_type(w, jnp.int32)


def _unpack_bf16_pair(w):
    u = jax.lax.bitcast_convert_type(w, jnp.uint32)
    lo = jax.lax.bitcast_convert_type(u << 16, jnp.float32)
    hi = jax.lax.bitcast_convert_type(u & jnp.uint32(0xFFFF0000), jnp.float32)
    return lo, hi


MAXC = 40


def _pad_worker_idx(idx):
    idx2d = idx.reshape(HCHUNKS, CH)
    per = HCHUNKS // NW
    rem = HCHUNKS % NW
    slabs = []
    for w in range(NW):
        b = w * per + min(w, rem)
        cnt = per + (1 if w < rem else 0)
        slabs.append(jnp.pad(idx2d[b:b + cnt], ((0, MAXC - cnt), (0, 0))))
    return jnp.stack(slabs)


def _worker_split(wid):
    per = HCHUNKS // NW
    rem = HCHUNKS % NW
    base = wid * per + jnp.minimum(wid, rem)
    cnt = per + jnp.where(wid < rem, 1, 0)
    return base, cnt


NBUF = 3


def _gather_add_body(p_hbm, q_hbm, row_hbm, col_hbm, z_hbm,
                     ridx, cidx, pbuf0, qbuf0, pbuf1, qbuf1, pbuf2, qbuf2,
                     isem, gsem0, gsem1, gsem2, wsem0, wsem1, wsem2):
    c = lax.axis_index("c")
    s = lax.axis_index("s")
    wid = s * NC + c
    base, cnt = _worker_split(wid)

    pltpu.async_copy(row_hbm.at[wid], ridx, isem)
    pltpu.async_copy(col_hbm.at[wid], cidx, isem)
    pltpu.make_async_copy(row_hbm.at[wid], ridx, isem).wait()
    pltpu.make_async_copy(col_hbm.at[wid], cidx, isem).wait()

    pbufs = (pbuf0, pbuf1, pbuf2)
    qbufs = (qbuf0, qbuf1, qbuf2)
    gsems = (gsem0, gsem1, gsem2)
    wsems = (wsem0, wsem1, wsem2)

    def _issue(j, b):
        pltpu.async_copy(p_hbm.at[ridx.at[j]], pbufs[b], gsems[b])
        pltpu.async_copy(q_hbm.at[cidx.at[j]], qbufs[b], gsems[b])

    def _process(j, b):
        pltpu.make_async_copy(p_hbm.at[ridx.at[j]], pbufs[b], gsems[b]).wait()
        pltpu.make_async_copy(q_hbm.at[cidx.at[j]], qbufs[b], gsems[b]).wait()
        pb, qb = pbufs[b], qbufs[b]

        def add_rows(i, carry):
            r = i * 4
            for rr in range(4):
                for cc in range(D // 16):
                    sl = pl.ds(cc * 16, 16)
                    pb[r + rr, sl] = pb[r + rr, sl] + qb[r + rr, sl]
            return carry

        lax.fori_loop(0, CH // 4, add_rows, 0)
        pltpu.async_copy(pb, z_hbm.at[pl.ds((base + j) * CH, CH)], wsems[b])

    def step(i, carry):
        for b in range(NBUF):
            j = i * NBUF + b

            @pl.when(j < cnt)
            def _():
                @pl.when(j >= NBUF)
                def _():
                    pltpu.make_async_copy(
                        pbufs[b], z_hbm.at[pl.ds(0, CH)], wsems[b]).wait()

                _issue(j, b)

            @pl.when((j >= 2) & (j <= cnt + 1))
            def _():
                _process(j - 2, (b + 1) % NBUF)
        return carry

    lax.fori_loop(0, (cnt + NBUF) // NBUF, step, 0)
    pltpu.make_async_copy(pbuf0, z_hbm.at[pl.ds(0, CH)], wsem0).wait()
    pltpu.make_async_copy(pbuf1, z_hbm.at[pl.ds(0, CH)], wsem1).wait()
    pltpu.make_async_copy(pbuf2, z_hbm.at[pl.ds(0, CH)], wsem2).wait()


@functools.lru_cache(maxsize=None)
def _gather_add_kernel():
    return pl.kernel(
        _gather_add_body,
        out_type=jax.ShapeDtypeStruct((E2, D), jnp.float32),
        mesh=_sc_mesh(),
        scratch_types=[
            pltpu.VMEM((MAXC, CH), jnp.int32),
            pltpu.VMEM((MAXC, CH), jnp.int32),
            pltpu.VMEM((CH, D), jnp.float32),
            pltpu.VMEM((CH, D), jnp.float32),
            pltpu.VMEM((CH, D), jnp.float32),
            pltpu.VMEM((CH, D), jnp.float32),
            pltpu.VMEM((CH, D), jnp.float32),
            pltpu.VMEM((CH, D), jnp.float32),
            pltpu.SemaphoreType.DMA,
            pltpu.SemaphoreType.DMA,
            pltpu.SemaphoreType.DMA,
            pltpu.SemaphoreType.DMA,
            pltpu.SemaphoreType.DMA,
            pltpu.SemaphoreType.DMA,
            pltpu.SemaphoreType.DMA,
        ],
    )


def _gather_add(p, q, rowp, colp):
    return _gather_add_kernel()(p, q, rowp, colp)


def _scatter_add_body(m_hbm, row_hbm, zeros_hbm, agg_hbm,
                      ridx, mbuf0, mbuf1, accum, lsem0, lsem1):
    c = lax.axis_index("c")
    s = lax.axis_index("s")
    wid = s * NC + c
    base, cnt = _worker_split(wid)
    pltpu.sync_copy(zeros_hbm.at[pl.ds(s * ROWS_PER_TILE, ROWS_PER_TILE)],
                    accum.at[pl.ds(s * ROWS_PER_TILE, ROWS_PER_TILE)])

    pltpu.sync_copy(row_hbm.at[wid], ridx)
    plsc.subcore_barrier()

    mbufs = (mbuf0, mbuf1)
    lsems = (lsem0, lsem1)

    def step(i, carry):
        for b in range(2):
            j = i * 2 + b

            @pl.when(j < cnt)
            def _():
                pltpu.async_copy(m_hbm.at[pl.ds((base + j) * CH, CH)],
                                 mbufs[b], lsems[b])

            @pl.when((j >= 1) & (j <= cnt))
            def _():
                pltpu.make_async_copy(
                    m_hbm.at[pl.ds(base * CH, CH)],
                    mbufs[1 - b], lsems[1 - b]).wait()
                pltpu.sync_copy(mbufs[1 - b], accum.at[ridx.at[j - 1]],
                                add=True)
        return carry

    lax.fori_loop(0, (cnt + 2) // 2, step, 0)
    plsc.subcore_barrier()
    r0 = s * ROWS_PER_TILE
    pltpu.sync_copy(accum.at[pl.ds(r0, ROWS_PER_TILE)],
                    agg_hbm.at[pl.ds(c * NP + r0, ROWS_PER_TILE)])


@functools.lru_cache(maxsize=None)
def _scatter_add_kernel():
    return pl.kernel(
        _scatter_add_body,
        out_type=jax.ShapeDtypeStruct((NC * NP, D), jnp.float32),
        mesh=_sc_mesh(),
        scratch_types=[
            pltpu.VMEM((MAXC, CH), jnp.int32),
            pltpu.VMEM((CH, D), jnp.float32),
            pltpu.VMEM((CH, D), jnp.float32),
            pltpu.VMEM_SHARED((NP, D), jnp.float32),
            pltpu.SemaphoreType.DMA,
            pltpu.SemaphoreType.DMA,
        ],
    )


def _scatter_add(m, rowp, zeros):
    return _scatter_add_kernel()(m, rowp, zeros)


BE = 4000
BN = 2000


def _edge_mlp_body(z_ref, ea_ref, ce_ref, co_ref, be_ref, bo_ref,
                   w2e_ref, w2o_ref, be2_ref, m_ref):
    z = z_ref[...]
    zlo = z[:, :D // 2]
    zhi = z[:, D // 2:]
    ea = ea_ref[...]
    ze = zlo + ea[:, 0:1] * ce_ref[0:1, :] + ea[:, 1:2] * ce_ref[1:2, :] + be_ref[...]
    zo = zhi + ea[:, 0:1] * co_ref[0:1, :] + ea[:, 1:2] * co_ref[1:2, :] + bo_ref[...]
    ae = _silu(ze)
    ao = _silu(zo)
    m = (jnp.dot(ae, w2e_ref[...], preferred_element_type=jnp.float32)
         + jnp.dot(ao, w2o_ref[...], preferred_element_type=jnp.float32)
         + be2_ref[...])
    m_ref[...] = _silu(m)


def _edge_mlp(z, ea, ce, co, be, bo, w2e, w2o, be2):
    grid = (E2 // BE,)
    H = D // 2
    return pl.pallas_call(
        _edge_mlp_body,
        grid=grid,
        in_specs=[
            pl.BlockSpec((BE, D), lambda i: (i, 0)),
            pl.BlockSpec((BE, 2), lambda i: (i, 0)),
            pl.BlockSpec((2, H), lambda i: (0, 0)),
            pl.BlockSpec((2, H), lambda i: (0, 0)),
            pl.BlockSpec((1, H), lambda i: (0, 0)),
            pl.BlockSpec((1, H), lambda i: (0, 0)),
            pl.BlockSpec((H, D), lambda i: (0, 0)),
            pl.BlockSpec((H, D), lambda i: (0, 0)),
            pl.BlockSpec((1, D), lambda i: (0, 0)),
        ],
        out_specs=pl.BlockSpec((BE, D), lambda i: (i, 0)),
        out_shape=jax.ShapeDtypeStruct((E2, D), jnp.float32),
    )(z, ea, ce, co, be, bo, w2e, w2o, be2)


def _node_body(h_ref, a0_ref, a1_ref, a2_ref, a3_ref, w1a_ref, w1b_ref,
               b1_ref, w2_ref, b2_ref, wpq_ref, h_out, p_out, q_out):
    h = h_ref[...]
    agg = (a0_ref[...] + a1_ref[...]) + (a2_ref[...] + a3_ref[...])
    u = (jnp.dot(h, w1a_ref[...], preferred_element_type=jnp.float32)
         + jnp.dot(agg, w1b_ref[...], preferred_element_type=jnp.float32)
         + b1_ref[...])
    u = _silu(u)
    hn = h + jnp.dot(u, w2_ref[...], preferred_element_type=jnp.float32) + b2_ref[...]
    h_out[...] = hn
    pq = jnp.dot(hn, wpq_ref[...], preferred_element_type=jnp.float32)
    p_out[...] = pq[:, :D]
    q_out[...] = pq[:, D:]


def _node_update(h, a0, a1, a2, a3, w1a, w1b, b1, w2, b2, wpq):
    grid = (N // BN,)
    full = lambda i: (0, 0)
    return pl.pallas_call(
        _node_body,
        grid=grid,
        in_specs=[
            pl.BlockSpec((BN, D), lambda i: (i, 0)),
            pl.BlockSpec((BN, D), lambda i: (i, 0)),
            pl.BlockSpec((BN, D), lambda i: (i, 0)),
            pl.BlockSpec((BN, D), lambda i: (i, 0)),
            pl.BlockSpec((BN, D), lambda i: (i, 0)),
            pl.BlockSpec((D, D), full),
            pl.BlockSpec((D, D), full),
            pl.BlockSpec((1, D), full),
            pl.BlockSpec((D, D), full),
            pl.BlockSpec((1, D), full),
            pl.BlockSpec((D, 2 * D), full),
        ],
        out_specs=[
            pl.BlockSpec((BN, D), lambda i: (i, 0)),
            pl.BlockSpec((BN, D), lambda i: (i, 0)),
            pl.BlockSpec((BN, D), lambda i: (i, 0)),
        ],
        out_shape=[
            jax.ShapeDtypeStruct((N, D), jnp.float32),
            jax.ShapeDtypeStruct((N, D), jnp.float32),
            jax.ShapeDtypeStruct((N, D), jnp.float32),
        ],
    )(h, a0, a1, a2, a3, w1a, w1b, b1, w2, b2, wpq)


def _embed_body(loc_ref, vel_ref, wl_ref, wv_ref, b_ref, wpq_ref,
                h_out, p_out, q_out):
    loc = loc_ref[...]
    vel = vel_ref[...]
    h = b_ref[...] + jnp.zeros((loc.shape[0], D), jnp.float32)
    for j in range(3):
        h = h + loc[:, j:j + 1] * wl_ref[j:j + 1, :]
        h = h + vel[:, j:j + 1] * wv_ref[j:j + 1, :]
    h_out[...] = h
    pq = jnp.dot(h, wpq_ref[...], preferred_element_type=jnp.float32)
    p_out[...] = pq[:, :D]
    q_out[...] = pq[:, D:]


def _embed(loc, vel, wl, wv, b, wpq):
    grid = (N // BN,)
    full = lambda i: (0, 0)
    return pl.pallas_call(
        _embed_body,
        grid=grid,
        in_specs=[
            pl.BlockSpec((BN, 3), lambda i: (i, 0)),
            pl.BlockSpec((BN, 3), lambda i: (i, 0)),
            pl.BlockSpec((3, D), full),
            pl.BlockSpec((3, D), full),
            pl.BlockSpec((1, D), full),
            pl.BlockSpec((D, 2 * D), full),
        ],
        out_specs=[
            pl.BlockSpec((BN, D), lambda i: (i, 0)),
            pl.BlockSpec((BN, D), lambda i: (i, 0)),
            pl.BlockSpec((BN, D), lambda i: (i, 0)),
        ],
        out_shape=[
            jax.ShapeDtypeStruct((N, D), jnp.float32),
            jax.ShapeDtypeStruct((N, D), jnp.float32),
            jax.ShapeDtypeStruct((N, D), jnp.float32),
        ],
    )(loc, vel, wl, wv, b, wpq)


def _node_decode_body(h_ref, a0_ref, a1_ref, a2_ref, a3_ref, w1a_ref,
                      w1b_ref, b1_ref, w2_ref, b2_ref, wd1_ref, bd1_ref,
                      wd2_ref, bd2_ref, o_ref):
    h = h_ref[...]
    agg = (a0_ref[...] + a1_ref[...]) + (a2_ref[...] + a3_ref[...])
    u = (jnp.dot(h, w1a_ref[...], preferred_element_type=jnp.float32)
         + jnp.dot(agg, w1b_ref[...], preferred_element_type=jnp.float32)
         + b1_ref[...])
    u = _silu(u)
    hn = h + jnp.dot(u, w2_ref[...], preferred_element_type=jnp.float32) + b2_ref[...]
    d = _silu(jnp.dot(hn, wd1_ref[...], preferred_element_type=jnp.float32)
              + bd1_ref[...])
    o_ref[...] = (jnp.dot(d, wd2_ref[...], preferred_element_type=jnp.float32)
                  + bd2_ref[...])


def _node_decode(h, a0, a1, a2, a3, w1a, w1b, b1, w2, b2, wd1, bd1, wd2, bd2):
    grid = (N // BN,)
    full = lambda i: (0, 0)
    blk = lambda i: (i, 0)
    return pl.pallas_call(
        _node_decode_body,
        grid=grid,
        in_specs=[
            pl.BlockSpec((BN, D), blk),
            pl.BlockSpec((BN, D), blk),
            pl.BlockSpec((BN, D), blk),
            pl.BlockSpec((BN, D), blk),
            pl.BlockSpec((BN, D), blk),
            pl.BlockSpec((D, D), full),
            pl.BlockSpec((D, D), full),
            pl.BlockSpec((1, D), full),
            pl.BlockSpec((D, D), full),
            pl.BlockSpec((1, D), full),
            pl.BlockSpec((D, D), full),
            pl.BlockSpec((1, D), full),
            pl.BlockSpec((D, 3), full),
            pl.BlockSpec((1, 3), full),
        ],
        out_specs=pl.BlockSpec((BN, 3), blk),
        out_shape=jax.ShapeDtypeStruct((N, 3), jnp.float32),
    )(h, a0, a1, a2, a3, w1a, w1b, b1, w2, b2, wd1, bd1, wd2, bd2)


def _decode_body(h_ref, w1_ref, b1_ref, w2_ref, b2_ref, o_ref):
    h = h_ref[...]
    d = _silu(jnp.dot(h, w1_ref[...], preferred_element_type=jnp.float32)
              + b1_ref[...])
    o_ref[...] = (jnp.dot(d, w2_ref[...], preferred_element_type=jnp.float32)
                  + b2_ref[...])


def _decode(h, w1, b1, w2, b2):
    grid = (N // BN,)
    full = lambda i: (0, 0)
    return pl.pallas_call(
        _decode_body,
        grid=grid,
        in_specs=[
            pl.BlockSpec((BN, D), lambda i: (i, 0)),
            pl.BlockSpec((D, D), full),
            pl.BlockSpec((1, D), full),
            pl.BlockSpec((D, 3), full),
            pl.BlockSpec((1, 3), full),
        ],
        out_specs=pl.BlockSpec((BN, 3), lambda i: (i, 0)),
        out_shape=jax.ShapeDtypeStruct((N, 3), jnp.float32),
    )(h, w1, b1, w2, b2)


def kernel(nodes, loc, edges, vel, edge_attr, _, W_emb, b_emb, We1, be1,
           We2, be2, Wn1, bn1, Wn2, bn2, Wd1, bd1, Wd2, bd2):
    row = edges[0]
    col = edges[1]
    rowp = [_pad_worker_idx(row[:E2]), _pad_worker_idx(row[E2:])]
    colp = [_pad_worker_idx(col[:E2]), _pad_worker_idx(col[E2:])]
    ea = [edge_attr[:E2], edge_attr[E2:]]
    zeros = jnp.zeros((NP, D), jnp.float32)

    def _wpq(i):
        A = We1[i, :D, :]
        B = We1[i, D:2 * D, :]
        return jnp.concatenate([A[:, 0::2], A[:, 1::2],
                                B[:, 0::2], B[:, 1::2]], axis=1)

    wpq = [_wpq(i) for i in range(L)]

    h, p, q = _embed(loc, vel, W_emb[:3], W_emb[3:], b_emb.reshape(1, D),
                     wpq[0])
    for i in range(L):
        c2 = We1[i, 2 * D:, :]
        ce = c2[:, 0::2]
        co = c2[:, 1::2]
        b1e = be1[i, 0::2].reshape(1, D // 2)
        b1o = be1[i, 1::2].reshape(1, D // 2)
        w2e = We2[i, 0::2, :]
        w2o = We2[i, 1::2, :]
        b2 = be2[i].reshape(1, D)
        z0 = _gather_add(p, q, rowp[0], colp[0])
        m0 = _edge_mlp(z0, ea[0], ce, co, b1e, b1o, w2e, w2o, b2)
        z1 = _gather_add(p, q, rowp[1], colp[1])
        agg0 = _scatter_add(m0, rowp[0], zeros)
        m1 = _edge_mlp(z1, ea[1], ce, co, b1e, b1o, w2e, w2o, b2)
        agg1 = _scatter_add(m1, rowp[1], zeros)
        if i < L - 1:
            h, p, q = _node_update(h, agg0[:N], agg0[NP:NP + N],
                                   agg1[:N], agg1[NP:NP + N],
                                   Wn1[i, :D, :], Wn1[i, D:, :],
                                   bn1[i].reshape(1, D), Wn2[i],
                                   bn2[i].reshape(1, D), wpq[i + 1])
        else:
            return _node_decode(h, agg0[:N], agg0[NP:NP + N],
                                agg1[:N], agg1[NP:NP + N],
                                Wn1[i, :D, :], Wn1[i, D:, :],
                                bn1[i].reshape(1, D), Wn2[i],
                                bn2[i].reshape(1, D),
                                Wd1, bd1.reshape(1, D), Wd2,
                                bd2.reshape(1, 3))


# speedup vs baseline: 1.0773x; 1.0756x over previous
import functools

import jax
import jax.numpy as jnp
from jax import lax
from jax.experimental import pallas as pl
from jax.experimental.pallas import tpu as pltpu
from jax.experimental.pallas import tpu_sc as plsc

N = 10000
E = 320000
D = 128
L = 4

NC = 2
NS = 16
NW = NC * NS
CH = 128
CHUNKS = E // CH
E2 = E // 2
HCHUNKS = E2 // CH
ROWS_PER_TILE = 632
NP = ROWS_PER_TILE * NS

@functools.lru_cache(maxsize=None)
def _sc_mesh():
    return plsc.VectorSubcoreMesh(
        core_axis_name="c", subcore_axis_name="s",
        num_cores=NC, num_subcores=NS)


def _silu(v):
    return v * (1.0 / (1.0 + jnp.exp(-v)))


def _rne_bf16_bits(x):
    rb = jax.lax.bitcast_convert_type(x, jnp.uint32)
    return (rb + jnp.uint32(0x7FFF) + ((rb >> 16) & jnp.uint32(1))) >> 16


def _pack_bf16_pair(lo, hi):
    w = _rne_bf16_bits(lo) | (_rne_bf16_bits(hi) << 16)
    return jax.lax.bitcast_convert_type(w, jnp.int32)


def _unpack_bf16_pair(w):
    u = jax.lax.bitcast_convert_type(w, jnp.uint32)
    lo = jax.lax.bitcast_convert_type(u << 16, jnp.float32)
    hi = jax.lax.bitcast_convert_type(u & jnp.uint32(0xFFFF0000), jnp.float32)
    return lo, hi


MAXC = 40


def _pad_worker_idx(idx):
    idx2d = idx.reshape(HCHUNKS, CH)
    per = HCHUNKS // NW
    rem = HCHUNKS % NW
    slabs = []
    for w in range(NW):
        b = w * per + min(w, rem)
        cnt = per + (1 if w < rem else 0)
        slabs.append(jnp.pad(idx2d[b:b + cnt], ((0, MAXC - cnt), (0, 0))))
    return jnp.stack(slabs)


def _worker_split(wid):
    per = HCHUNKS // NW
    rem = HCHUNKS % NW
    base = wid * per + jnp.minimum(wid, rem)
    cnt = per + jnp.where(wid < rem, 1, 0)
    return base, cnt


NBUF = 3


def _gather_add_body(p_hbm, q_hbm, row_hbm, col_hbm, z_hbm,
                     ridx, cidx, pbuf0, qbuf0, pbuf1, qbuf1, pbuf2, qbuf2,
                     isem, gsem0, gsem1, gsem2, wsem0, wsem1, wsem2):
    c = lax.axis_index("c")
    s = lax.axis_index("s")
    wid = s * NC + c
    base, cnt = _worker_split(wid)

    pltpu.async_copy(row_hbm.at[wid], ridx, isem)
    pltpu.async_copy(col_hbm.at[wid], cidx, isem)
    pltpu.make_async_copy(row_hbm.at[wid], ridx, isem).wait()
    pltpu.make_async_copy(col_hbm.at[wid], cidx, isem).wait()

    pbufs = (pbuf0, pbuf1, pbuf2)
    qbufs = (qbuf0, qbuf1, qbuf2)
    gsems = (gsem0, gsem1, gsem2)
    wsems = (wsem0, wsem1, wsem2)

    def _issue(j, b):
        pltpu.async_copy(p_hbm.at[ridx.at[j]], pbufs[b], gsems[b])
        pltpu.async_copy(q_hbm.at[cidx.at[j]], qbufs[b], gsems[b])

    def _process(j, b):
        pltpu.make_async_copy(p_hbm.at[ridx.at[j]], pbufs[b], gsems[b]).wait()
        pltpu.make_async_copy(q_hbm.at[cidx.at[j]], qbufs[b], gsems[b]).wait()
        pb, qb = pbufs[b], qbufs[b]

        def add_rows(i, carry):
            r = i * 4
            for rr in range(4):
                for cc in range(D // 16):
                    sl = pl.ds(cc * 16, 16)
                    pb[r + rr, sl] = pb[r + rr, sl] + qb[r + rr, sl]
            return carry

        lax.fori_loop(0, CH // 4, add_rows, 0)
        pltpu.async_copy(pb, z_hbm.at[pl.ds((base + j) * CH, CH)], wsems[b])

    def step(i, carry):
        for b in range(NBUF):
            j = i * NBUF + b

            @pl.when(j < cnt)
            def _():
                @pl.when(j >= NBUF)
                def _():
                    pltpu.make_async_copy(
                        pbufs[b], z_hbm.at[pl.ds(0, CH)], wsems[b]).wait()

                _issue(j, b)

            @pl.when((j >= 2) & (j <= cnt + 1))
            def _():
                _process(j - 2, (b + 1) % NBUF)
        return carry

    lax.fori_loop(0, (cnt + NBUF) // NBUF, step, 0)
    pltpu.make_async_copy(pbuf0, z_hbm.at[pl.ds(0, CH)], wsem0).wait()
    pltpu.make_async_copy(pbuf1, z_hbm.at[pl.ds(0, CH)], wsem1).wait()
    pltpu.make_async_copy(pbuf2, z_hbm.at[pl.ds(0, CH)], wsem2).wait()


@functools.lru_cache(maxsize=None)
def _gather_add_kernel():
    return pl.kernel(
        _gather_add_body,
        out_type=jax.ShapeDtypeStruct((E2, D), jnp.float32),
        mesh=_sc_mesh(),
        scratch_types=[
            pltpu.VMEM((MAXC, CH), jnp.int32),
            pltpu.VMEM((MAXC, CH), jnp.int32),
            pltpu.VMEM((CH, D), jnp.float32),
            pltpu.VMEM((CH, D), jnp.float32),
            pltpu.VMEM((CH, D), jnp.float32),
            pltpu.VMEM((CH, D), jnp.float32),
            pltpu.VMEM((CH, D), jnp.float32),
            pltpu.VMEM((CH, D), jnp.float32),
            pltpu.SemaphoreType.DMA,
            pltpu.SemaphoreType.DMA,
            pltpu.SemaphoreType.DMA,
            pltpu.SemaphoreType.DMA,
            pltpu.SemaphoreType.DMA,
            pltpu.SemaphoreType.DMA,
            pltpu.SemaphoreType.DMA,
        ],
    )


def _gather_add(p, q, rowp, colp):
    return _gather_add_kernel()(p, q, rowp, colp)


def _scatter_add_body(m_hbm, row_hbm, zeros_hbm, agg_hbm,
                      ridx, mbuf0, mbuf1, accum, lsem0, lsem1):
    c = lax.axis_index("c")
    s = lax.axis_index("s")
    wid = s * NC + c
    base, cnt = _worker_split(wid)
    pltpu.sync_copy(zeros_hbm.at[pl.ds(s * ROWS_PER_TILE, ROWS_PER_TILE)],
                    accum.at[pl.ds(s * ROWS_PER_TILE, ROWS_PER_TILE)])

    pltpu.sync_copy(row_hbm.at[wid], ridx)
    plsc.subcore_barrier()

    mbufs = (mbuf0, mbuf1)
    lsems = (lsem0, lsem1)

    def step(i, carry):
        for b in range(2):
            j = i * 2 + b

            @pl.when(j < cnt)
            def _():
                pltpu.async_copy(m_hbm.at[pl.ds((base + j) * CH, CH)],
                                 mbufs[b], lsems[b])

            @pl.when((j >= 1) & (j <= cnt))
            def _():
                pltpu.make_async_copy(
                    m_hbm.at[pl.ds(base * CH, CH)],
                    mbufs[1 - b], lsems[1 - b]).wait()
                pltpu.sync_copy(mbufs[1 - b], accum.at[ridx.at[j - 1]],
                                add=True)
        return carry

    lax.fori_loop(0, (cnt + 2) // 2, step, 0)
    plsc.subcore_barrier()
    r0 = s * ROWS_PER_TILE
    pltpu.sync_copy(accum.at[pl.ds(r0, ROWS_PER_TILE)],
                    agg_hbm.at[pl.ds(c * NP + r0, ROWS_PER_TILE)])


@functools.lru_cache(maxsize=None)
def _scatter_add_kernel():
    return pl.kernel(
        _scatter_add_body,
        out_type=jax.ShapeDtypeStruct((NC * NP, D), jnp.float32),
        mesh=_sc_mesh(),
        scratch_types=[
            pltpu.VMEM((MAXC, CH), jnp.int32),
            pltpu.VMEM((CH, D), jnp.float32),
            pltpu.VMEM((CH, D), jnp.float32),
            pltpu.VMEM_SHARED((NP, D), jnp.float32),
            pltpu.SemaphoreType.DMA,
            pltpu.SemaphoreType.DMA,
        ],
    )


def _scatter_add(m, rowp, zeros):
    return _scatter_add_kernel()(m, rowp, zeros)


BE = 4000
BN = 2000


def _edge_mlp_body(z_ref, ea_ref, c_ref, be1_ref, w2_ref, be2_ref, m_ref):
    z = z_ref[...]
    ea = ea_ref[...]
    z = z + ea[:, 0:1] * c_ref[0:1, :] + ea[:, 1:2] * c_ref[1:2, :] + be1_ref[...]
    a = _silu(z)
    m = jnp.dot(a, w2_ref[...], preferred_element_type=jnp.float32) + be2_ref[...]
    m_ref[...] = _silu(m)


def _edge_mlp(z, ea, c2, be1, w2, be2):
    grid = (E2 // BE,)
    return pl.pallas_call(
        _edge_mlp_body,
        grid=grid,
        in_specs=[
            pl.BlockSpec((BE, D), lambda i: (i, 0)),
            pl.BlockSpec((BE, 2), lambda i: (i, 0)),
            pl.BlockSpec((2, D), lambda i: (0, 0)),
            pl.BlockSpec((1, D), lambda i: (0, 0)),
            pl.BlockSpec((D, D), lambda i: (0, 0)),
            pl.BlockSpec((1, D), lambda i: (0, 0)),
        ],
        out_specs=pl.BlockSpec((BE, D), lambda i: (i, 0)),
        out_shape=jax.ShapeDtypeStruct((E2, D), jnp.float32),
    )(z, ea, c2, be1, w2, be2)


def _node_body(h_ref, a0_ref, a1_ref, a2_ref, a3_ref, w1a_ref, w1b_ref,
               b1_ref, w2_ref, b2_ref, wpq_ref, h_out, p_out, q_out):
    h = h_ref[...]
    agg = (a0_ref[...] + a1_ref[...]) + (a2_ref[...] + a3_ref[...])
    u = (jnp.dot(h, w1a_ref[...], preferred_element_type=jnp.float32)
         + jnp.dot(agg, w1b_ref[...], preferred_element_type=jnp.float32)
         + b1_ref[...])
    u = _silu(u)
    hn = h + jnp.dot(u, w2_ref[...], preferred_element_type=jnp.float32) + b2_ref[...]
    h_out[...] = hn
    pq = jnp.dot(hn, wpq_ref[...], preferred_element_type=jnp.float32)
    p_out[...] = pq[:, :D]
    q_out[...] = pq[:, D:]


def _node_update(h, a0, a1, a2, a3, w1a, w1b, b1, w2, b2, wpq):
    grid = (N // BN,)
    full = lambda i: (0, 0)
    return pl.pallas_call(
        _node_body,
        grid=grid,
        in_specs=[
            pl.BlockSpec((BN, D), lambda i: (i, 0)),
            pl.BlockSpec((BN, D), lambda i: (i, 0)),
            pl.BlockSpec((BN, D), lambda i: (i, 0)),
            pl.BlockSpec((BN, D), lambda i: (i, 0)),
            pl.BlockSpec((BN, D), lambda i: (i, 0)),
            pl.BlockSpec((D, D), full),
            pl.BlockSpec((D, D), full),
            pl.BlockSpec((1, D), full),
            pl.BlockSpec((D, D), full),
            pl.BlockSpec((1, D), full),
            pl.BlockSpec((D, 2 * D), full),
        ],
        out_specs=[
            pl.BlockSpec((BN, D), lambda i: (i, 0)),
            pl.BlockSpec((BN, D), lambda i: (i, 0)),
            pl.BlockSpec((BN, D), lambda i: (i, 0)),
        ],
        out_shape=[
            jax.ShapeDtypeStruct((N, D), jnp.float32),
            jax.ShapeDtypeStruct((N, D), jnp.float32),
            jax.ShapeDtypeStruct((N, D), jnp.float32),
        ],
    )(h, a0, a1, a2, a3, w1a, w1b, b1, w2, b2, wpq)


def _embed_body(loc_ref, vel_ref, wl_ref, wv_ref, b_ref, wpq_ref,
                h_out, p_out, q_out):
    loc = loc_ref[...]
    vel = vel_ref[...]
    h = b_ref[...] + jnp.zeros((loc.shape[0], D), jnp.float32)
    for j in range(3):
        h = h + loc[:, j:j + 1] * wl_ref[j:j + 1, :]
        h = h + vel[:, j:j + 1] * wv_ref[j:j + 1, :]
    h_out[...] = h
    pq = jnp.dot(h, wpq_ref[...], preferred_element_type=jnp.float32)
    p_out[...] = pq[:, :D]
    q_out[...] = pq[:, D:]


def _embed(loc, vel, wl, wv, b, wpq):
    grid = (N // BN,)
    full = lambda i: (0, 0)
    return pl.pallas_call(
        _embed_body,
        grid=grid,
        in_specs=[
            pl.BlockSpec((BN, 3), lambda i: (i, 0)),
            pl.BlockSpec((BN, 3), lambda i: (i, 0)),
            pl.BlockSpec((3, D), full),
            pl.BlockSpec((3, D), full),
            pl.BlockSpec((1, D), full),
            pl.BlockSpec((D, 2 * D), full),
        ],
        out_specs=[
            pl.BlockSpec((BN, D), lambda i: (i, 0)),
            pl.BlockSpec((BN, D), lambda i: (i, 0)),
            pl.BlockSpec((BN, D), lambda i: (i, 0)),
        ],
        out_shape=[
            jax.ShapeDtypeStruct((N, D), jnp.float32),
            jax.ShapeDtypeStruct((N, D), jnp.float32),
            jax.ShapeDtypeStruct((N, D), jnp.float32),
        ],
    )(loc, vel, wl, wv, b, wpq)


def _node_decode_body(h_ref, a0_ref, a1_ref, a2_ref, a3_ref, w1a_ref,
                      w1b_ref, b1_ref, w2_ref, b2_ref, wd1_ref, bd1_ref,
                      wd2_ref, bd2_ref, o_ref):
    h = h_ref[...]
    agg = (a0_ref[...] + a1_ref[...]) + (a2_ref[...] + a3_ref[...])
    u = (jnp.dot(h, w1a_ref[...], preferred_element_type=jnp.float32)
         + jnp.dot(agg, w1b_ref[...], preferred_element_type=jnp.float32)
         + b1_ref[...])
    u = _silu(u)
    hn = h + jnp.dot(u, w2_ref[...], preferred_element_type=jnp.float32) + b2_ref[...]
    d = _silu(jnp.dot(hn, wd1_ref[...], preferred_element_type=jnp.float32)
              + bd1_ref[...])
    o_ref[...] = (jnp.dot(d, wd2_ref[...], preferred_element_type=jnp.float32)
                  + bd2_ref[...])


def _node_decode(h, a0, a1, a2, a3, w1a, w1b, b1, w2, b2, wd1, bd1, wd2, bd2):
    grid = (N // BN,)
    full = lambda i: (0, 0)
    blk = lambda i: (i, 0)
    return pl.pallas_call(
        _node_decode_body,
        grid=grid,
        in_specs=[
            pl.BlockSpec((BN, D), blk),
            pl.BlockSpec((BN, D), blk),
            pl.BlockSpec((BN, D), blk),
            pl.BlockSpec((BN, D), blk),
            pl.BlockSpec((BN, D), blk),
            pl.BlockSpec((D, D), full),
            pl.BlockSpec((D, D), full),
            pl.BlockSpec((1, D), full),
            pl.BlockSpec((D, D), full),
            pl.BlockSpec((1, D), full),
            pl.BlockSpec((D, D), full),
            pl.BlockSpec((1, D), full),
            pl.BlockSpec((D, 3), full),
            pl.BlockSpec((1, 3), full),
        ],
        out_specs=pl.BlockSpec((BN, 3), blk),
        out_shape=jax.ShapeDtypeStruct((N, 3), jnp.float32),
    )(h, a0, a1, a2, a3, w1a, w1b, b1, w2, b2, wd1, bd1, wd2, bd2)


def _decode_body(h_ref, w1_ref, b1_ref, w2_ref, b2_ref, o_ref):
    h = h_ref[...]
    d = _silu(jnp.dot(h, w1_ref[...], preferred_element_type=jnp.float32)
              + b1_ref[...])
    o_ref[...] = (jnp.dot(d, w2_ref[...], preferred_element_type=jnp.float32)
                  + b2_ref[...])


def _decode(h, w1, b1, w2, b2):
    grid = (N // BN,)
    full = lambda i: (0, 0)
    return pl.pallas_call(
        _decode_body,
        grid=grid,
        in_specs=[
            pl.BlockSpec((BN, D), lambda i: (i, 0)),
            pl.BlockSpec((D, D), full),
            pl.BlockSpec((1, D), full),
            pl.BlockSpec((D, 3), full),
            pl.BlockSpec((1, 3), full),
        ],
        out_specs=pl.BlockSpec((BN, 3), lambda i: (i, 0)),
        out_shape=jax.ShapeDtypeStruct((N, 3), jnp.float32),
    )(h, w1, b1, w2, b2)


def kernel(nodes, loc, edges, vel, edge_attr, _, W_emb, b_emb, We1, be1,
           We2, be2, Wn1, bn1, Wn2, bn2, Wd1, bd1, Wd2, bd2):
    row = edges[0]
    col = edges[1]
    rowp = [_pad_worker_idx(row[:E2]), _pad_worker_idx(row[E2:])]
    colp = [_pad_worker_idx(col[:E2]), _pad_worker_idx(col[E2:])]
    ea = [edge_attr[:E2], edge_attr[E2:]]
    zeros = jnp.zeros((NP, D), jnp.float32)

    def _wpq(i):
        A = We1[i, :D, :]
        B = We1[i, D:2 * D, :]
        return jnp.concatenate([A[:, 0::2], A[:, 1::2],
                                B[:, 0::2], B[:, 1::2]], axis=1)

    wpq = [_wpq(i) for i in range(L)]

    h, p, q = _embed(loc, vel, W_emb[:3], W_emb[3:], b_emb.reshape(1, D),
                     wpq[0])
    permi = jnp.concatenate([jnp.arange(0, D, 2), jnp.arange(1, D, 2)])
    for i in range(L):
        c2 = We1[i, 2 * D:, :][:, permi]
        b1 = be1[i][permi].reshape(1, D)
        w2 = We2[i][permi, :]
        b2 = be2[i].reshape(1, D)
        z0 = _gather_add(p, q, rowp[0], colp[0])
        m0 = _edge_mlp(z0, ea[0], c2, b1, w2, b2)
        z1 = _gather_add(p, q, rowp[1], colp[1])
        agg0 = _scatter_add(m0, rowp[0], zeros)
        m1 = _edge_mlp(z1, ea[1], c2, b1, w2, b2)
        agg1 = _scatter_add(m1, rowp[1], zeros)
        if i < L - 1:
            h, p, q = _node_update(h, agg0[:N], agg0[NP:NP + N],
                                   agg1[:N], agg1[NP:NP + N],
                                   Wn1[i, :D, :], Wn1[i, D:, :],
                                   bn1[i].reshape(1, D), Wn2[i],
                                   bn2[i].reshape(1, D), wpq[i + 1])
        else:
            return _node_decode(h, agg0[:N], agg0[NP:NP + N],
                                agg1[:N], agg1[NP:NP + N],
                                Wn1[i, :D, :], Wn1[i, D:, :],
                                bn1[i].reshape(1, D), Wn2[i],
                                bn2[i].reshape(1, D),
                                Wd1, bd1.reshape(1, D), Wd2,
                                bd2.reshape(1, 3))


# speedup vs baseline: 1.0786x; 1.0011x over previous
import functools

import jax
import jax.numpy as jnp
from jax import lax
from jax.experimental import pallas as pl
from jax.experimental.pallas import tpu as pltpu
from jax.experimental.pallas import tpu_sc as plsc

N = 10000
E = 320000
D = 128
L = 4

NC = 2
NS = 16
NW = NC * NS
CH = 128
CHUNKS = E // CH
E2 = E // 2
HCHUNKS = E2 // CH
ROWS_PER_TILE = 632
NP = ROWS_PER_TILE * NS

@functools.lru_cache(maxsize=None)
def _sc_mesh():
    return plsc.VectorSubcoreMesh(
        core_axis_name="c", subcore_axis_name="s",
        num_cores=NC, num_subcores=NS)


def _silu(v):
    return v * (1.0 / (1.0 + jnp.exp(-v)))


def _rne_bf16_bits(x):
    rb = jax.lax.bitcast_convert_type(x, jnp.uint32)
    return (rb + jnp.uint32(0x7FFF) + ((rb >> 16) & jnp.uint32(1))) >> 16


def _pack_bf16_pair(lo, hi):
    w = _rne_bf16_bits(lo) | (_rne_bf16_bits(hi) << 16)
    return jax.lax.bitcast_convert_type(w, jnp.int32)


def _unpack_bf16_pair(w):
    u = jax.lax.bitcast_convert_type(w, jnp.uint32)
    lo = jax.lax.bitcast_convert_type(u << 16, jnp.float32)
    hi = jax.lax.bitcast_convert_type(u & jnp.uint32(0xFFFF0000), jnp.float32)
    return lo, hi


MAXC = 40


def _pad_worker_idx(idx):
    idx2d = idx.reshape(HCHUNKS, CH)
    per = HCHUNKS // NW
    rem = HCHUNKS % NW
    slabs = []
    for w in range(NW):
        b = w * per + min(w, rem)
        cnt = per + (1 if w < rem else 0)
        slabs.append(jnp.pad(idx2d[b:b + cnt], ((0, MAXC - cnt), (0, 0))))
    return jnp.stack(slabs)


def _worker_split(wid):
    per = HCHUNKS // NW
    rem = HCHUNKS % NW
    base = wid * per + jnp.minimum(wid, rem)
    cnt = per + jnp.where(wid < rem, 1, 0)
    return base, cnt


NBUF = 3


def _gather_add_body(p_hbm, q_hbm, row_hbm, col_hbm, z_hbm,
                     ridx, cidx, pbuf0, qbuf0, pbuf1, qbuf1, pbuf2, qbuf2,
                     isem, gsem0, gsem1, gsem2, wsem0, wsem1, wsem2):
    c = lax.axis_index("c")
    s = lax.axis_index("s")
    wid = s * NC + c
    base, cnt = _worker_split(wid)

    pltpu.async_copy(row_hbm.at[wid], ridx, isem)
    pltpu.async_copy(col_hbm.at[wid], cidx, isem)
    pltpu.make_async_copy(row_hbm.at[wid], ridx, isem).wait()
    pltpu.make_async_copy(col_hbm.at[wid], cidx, isem).wait()

    pbufs = (pbuf0, pbuf1, pbuf2)
    qbufs = (qbuf0, qbuf1, qbuf2)
    gsems = (gsem0, gsem1, gsem2)
    wsems = (wsem0, wsem1, wsem2)

    def _issue(j, b):
        pltpu.async_copy(p_hbm.at[ridx.at[j]], pbufs[b], gsems[b])
        pltpu.async_copy(q_hbm.at[cidx.at[j]], qbufs[b], gsems[b])

    def _process(j, b):
        pltpu.make_async_copy(p_hbm.at[ridx.at[j]], pbufs[b], gsems[b]).wait()
        pltpu.make_async_copy(q_hbm.at[cidx.at[j]], qbufs[b], gsems[b]).wait()
        pb, qb = pbufs[b], qbufs[b]

        def add_rows(i, carry):
            r = i * 4
            for rr in range(4):
                for cc in range(D // 16):
                    sl = pl.ds(cc * 16, 16)
                    pb[r + rr, sl] = pb[r + rr, sl] + qb[r + rr, sl]
            return carry

        lax.fori_loop(0, CH // 4, add_rows, 0)
        pltpu.async_copy(pb, z_hbm.at[pl.ds((base + j) * CH, CH)], wsems[b])

    def step(i, carry):
        for b in range(NBUF):
            j = i * NBUF + b

            @pl.when(j < cnt)
            def _():
                @pl.when(j >= NBUF)
                def _():
                    pltpu.make_async_copy(
                        pbufs[b], z_hbm.at[pl.ds(0, CH)], wsems[b]).wait()

                _issue(j, b)

            @pl.when((j >= 2) & (j <= cnt + 1))
            def _():
                _process(j - 2, (b + 1) % NBUF)
        return carry

    lax.fori_loop(0, (cnt + NBUF) // NBUF, step, 0)
    pltpu.make_async_copy(pbuf0, z_hbm.at[pl.ds(0, CH)], wsem0).wait()
    pltpu.make_async_copy(pbuf1, z_hbm.at[pl.ds(0, CH)], wsem1).wait()
    pltpu.make_async_copy(pbuf2, z_hbm.at[pl.ds(0, CH)], wsem2).wait()


@functools.lru_cache(maxsize=None)
def _gather_add_kernel():
    return pl.kernel(
        _gather_add_body,
        out_type=jax.ShapeDtypeStruct((E2, D), jnp.float32),
        mesh=_sc_mesh(),
        scratch_types=[
            pltpu.VMEM((MAXC, CH), jnp.int32),
            pltpu.VMEM((MAXC, CH), jnp.int32),
            pltpu.VMEM((CH, D), jnp.float32),
            pltpu.VMEM((CH, D), jnp.float32),
            pltpu.VMEM((CH, D), jnp.float32),
            pltpu.VMEM((CH, D), jnp.float32),
            pltpu.VMEM((CH, D), jnp.float32),
            pltpu.VMEM((CH, D), jnp.float32),
            pltpu.SemaphoreType.DMA,
            pltpu.SemaphoreType.DMA,
            pltpu.SemaphoreType.DMA,
            pltpu.SemaphoreType.DMA,
            pltpu.SemaphoreType.DMA,
            pltpu.SemaphoreType.DMA,
            pltpu.SemaphoreType.DMA,
        ],
    )


def _gather_add(p, q, rowp, colp):
    return _gather_add_kernel()(p, q, rowp, colp)


def _scatter_add_body(m_hbm, row_hbm, zeros_hbm, agg_hbm,
                      ridx, mbuf0, mbuf1, accum, lsem0, lsem1):
    c = lax.axis_index("c")
    s = lax.axis_index("s")
    wid = s * NC + c
    base, cnt = _worker_split(wid)
    zsrc = zeros_hbm.at[pl.ds(s * ROWS_PER_TILE, ROWS_PER_TILE)]
    zdst = accum.at[pl.ds(s * ROWS_PER_TILE, ROWS_PER_TILE)]
    pltpu.async_copy(zsrc, zdst, lsem0)
    pltpu.async_copy(row_hbm.at[wid], ridx, lsem1)
    pltpu.make_async_copy(zsrc, zdst, lsem0).wait()
    pltpu.make_async_copy(row_hbm.at[wid], ridx, lsem1).wait()
    plsc.subcore_barrier()

    mbufs = (mbuf0, mbuf1)
    lsems = (lsem0, lsem1)

    def step(i, carry):
        for b in range(2):
            j = i * 2 + b

            @pl.when(j < cnt)
            def _():
                pltpu.async_copy(m_hbm.at[pl.ds((base + j) * CH, CH)],
                                 mbufs[b], lsems[b])

            @pl.when((j >= 1) & (j <= cnt))
            def _():
                pltpu.make_async_copy(
                    m_hbm.at[pl.ds(base * CH, CH)],
                    mbufs[1 - b], lsems[1 - b]).wait()
                pltpu.sync_copy(mbufs[1 - b], accum.at[ridx.at[j - 1]],
                                add=True)
        return carry

    lax.fori_loop(0, (cnt + 2) // 2, step, 0)
    plsc.subcore_barrier()
    r0 = s * ROWS_PER_TILE
    pltpu.sync_copy(accum.at[pl.ds(r0, ROWS_PER_TILE)],
                    agg_hbm.at[pl.ds(c * NP + r0, ROWS_PER_TILE)])


@functools.lru_cache(maxsize=None)
def _scatter_add_kernel():
    return pl.kernel(
        _scatter_add_body,
        out_type=jax.ShapeDtypeStruct((NC * NP, D), jnp.float32),
        mesh=_sc_mesh(),
        scratch_types=[
            pltpu.VMEM((MAXC, CH), jnp.int32),
            pltpu.VMEM((CH, D), jnp.float32),
            pltpu.VMEM((CH, D), jnp.float32),
            pltpu.VMEM_SHARED((NP, D), jnp.float32),
            pltpu.SemaphoreType.DMA,
            pltpu.SemaphoreType.DMA,
        ],
    )


def _scatter_add(m, rowp, zeros):
    return _scatter_add_kernel()(m, rowp, zeros)


BE = 4000
BN = 2000


def _edge_mlp_body(z_ref, ea_ref, c_ref, be1_ref, w2_ref, be2_ref, m_ref):
    z = z_ref[...]
    ea = ea_ref[...]
    z = z + ea[:, 0:1] * c_ref[0:1, :] + ea[:, 1:2] * c_ref[1:2, :] + be1_ref[...]
    a = _silu(z)
    m = jnp.dot(a, w2_ref[...], preferred_element_type=jnp.float32) + be2_ref[...]
    m_ref[...] = _silu(m)


def _edge_mlp(z, ea, c2, be1, w2, be2):
    grid = (E2 // BE,)
    return pl.pallas_call(
        _edge_mlp_body,
        grid=grid,
        in_specs=[
            pl.BlockSpec((BE, D), lambda i: (i, 0)),
            pl.BlockSpec((BE, 2), lambda i: (i, 0)),
            pl.BlockSpec((2, D), lambda i: (0, 0)),
            pl.BlockSpec((1, D), lambda i: (0, 0)),
            pl.BlockSpec((D, D), lambda i: (0, 0)),
            pl.BlockSpec((1, D), lambda i: (0, 0)),
        ],
        out_specs=pl.BlockSpec((BE, D), lambda i: (i, 0)),
        out_shape=jax.ShapeDtypeStruct((E2, D), jnp.float32),
    )(z, ea, c2, be1, w2, be2)


def _node_body(h_ref, a0_ref, a1_ref, a2_ref, a3_ref, w1a_ref, w1b_ref,
               b1_ref, w2_ref, b2_ref, wpq_ref, h_out, p_out, q_out):
    h = h_ref[...]
    agg = (a0_ref[...] + a1_ref[...]) + (a2_ref[...] + a3_ref[...])
    u = (jnp.dot(h, w1a_ref[...], preferred_element_type=jnp.float32)
         + jnp.dot(agg, w1b_ref[...], preferred_element_type=jnp.float32)
         + b1_ref[...])
    u = _silu(u)
    hn = h + jnp.dot(u, w2_ref[...], preferred_element_type=jnp.float32) + b2_ref[...]
    h_out[...] = hn
    pq = jnp.dot(hn, wpq_ref[...], preferred_element_type=jnp.float32)
    p_out[...] = pq[:, :D]
    q_out[...] = pq[:, D:]


def _node_update(h, a0, a1, a2, a3, w1a, w1b, b1, w2, b2, wpq):
    grid = (N // BN,)
    full = lambda i: (0, 0)
    return pl.pallas_call(
        _node_body,
        grid=grid,
        in_specs=[
            pl.BlockSpec((BN, D), lambda i: (i, 0)),
            pl.BlockSpec((BN, D), lambda i: (i, 0)),
            pl.BlockSpec((BN, D), lambda i: (i, 0)),
            pl.BlockSpec((BN, D), lambda i: (i, 0)),
            pl.BlockSpec((BN, D), lambda i: (i, 0)),
            pl.BlockSpec((D, D), full),
            pl.BlockSpec((D, D), full),
            pl.BlockSpec((1, D), full),
            pl.BlockSpec((D, D), full),
            pl.BlockSpec((1, D), full),
            pl.BlockSpec((D, 2 * D), full),
        ],
        out_specs=[
            pl.BlockSpec((BN, D), lambda i: (i, 0)),
            pl.BlockSpec((BN, D), lambda i: (i, 0)),
            pl.BlockSpec((BN, D), lambda i: (i, 0)),
        ],
        out_shape=[
            jax.ShapeDtypeStruct((N, D), jnp.float32),
            jax.ShapeDtypeStruct((N, D), jnp.float32),
            jax.ShapeDtypeStruct((N, D), jnp.float32),
        ],
    )(h, a0, a1, a2, a3, w1a, w1b, b1, w2, b2, wpq)


def _embed_body(loc_ref, vel_ref, wl_ref, wv_ref, b_ref, wpq_ref,
                h_out, p_out, q_out):
    loc = loc_ref[...]
    vel = vel_ref[...]
    h = b_ref[...] + jnp.zeros((loc.shape[0], D), jnp.float32)
    for j in range(3):
        h = h + loc[:, j:j + 1] * wl_ref[j:j + 1, :]
        h = h + vel[:, j:j + 1] * wv_ref[j:j + 1, :]
    h_out[...] = h
    pq = jnp.dot(h, wpq_ref[...], preferred_element_type=jnp.float32)
    p_out[...] = pq[:, :D]
    q_out[...] = pq[:, D:]


def _embed(loc, vel, wl, wv, b, wpq):
    grid = (N // BN,)
    full = lambda i: (0, 0)
    return pl.pallas_call(
        _embed_body,
        grid=grid,
        in_specs=[
            pl.BlockSpec((BN, 3), lambda i: (i, 0)),
            pl.BlockSpec((BN, 3), lambda i: (i, 0)),
            pl.BlockSpec((3, D), full),
            pl.BlockSpec((3, D), full),
            pl.BlockSpec((1, D), full),
            pl.BlockSpec((D, 2 * D), full),
        ],
        out_specs=[
            pl.BlockSpec((BN, D), lambda i: (i, 0)),
            pl.BlockSpec((BN, D), lambda i: (i, 0)),
            pl.BlockSpec((BN, D), lambda i: (i, 0)),
        ],
        out_shape=[
            jax.ShapeDtypeStruct((N, D), jnp.float32),
            jax.ShapeDtypeStruct((N, D), jnp.float32),
            jax.ShapeDtypeStruct((N, D), jnp.float32),
        ],
    )(loc, vel, wl, wv, b, wpq)


def _node_decode_body(h_ref, a0_ref, a1_ref, a2_ref, a3_ref, w1a_ref,
                      w1b_ref, b1_ref, w2_ref, b2_ref, wd1_ref, bd1_ref,
                      wd2_ref, bd2_ref, o_ref):
    h = h_ref[...]
    agg = (a0_ref[...] + a1_ref[...]) + (a2_ref[...] + a3_ref[...])
    u = (jnp.dot(h, w1a_ref[...], preferred_element_type=jnp.float32)
         + jnp.dot(agg, w1b_ref[...], preferred_element_type=jnp.float32)
         + b1_ref[...])
    u = _silu(u)
    hn = h + jnp.dot(u, w2_ref[...], preferred_element_type=jnp.float32) + b2_ref[...]
    d = _silu(jnp.dot(hn, wd1_ref[...], preferred_element_type=jnp.float32)
              + bd1_ref[...])
    o_ref[...] = (jnp.dot(d, wd2_ref[...], preferred_element_type=jnp.float32)
                  + bd2_ref[...])


def _node_decode(h, a0, a1, a2, a3, w1a, w1b, b1, w2, b2, wd1, bd1, wd2, bd2):
    grid = (N // BN,)
    full = lambda i: (0, 0)
    blk = lambda i: (i, 0)
    return pl.pallas_call(
        _node_decode_body,
        grid=grid,
        in_specs=[
            pl.BlockSpec((BN, D), blk),
            pl.BlockSpec((BN, D), blk),
            pl.BlockSpec((BN, D), blk),
            pl.BlockSpec((BN, D), blk),
            pl.BlockSpec((BN, D), blk),
            pl.BlockSpec((D, D), full),
            pl.BlockSpec((D, D), full),
            pl.BlockSpec((1, D), full),
            pl.BlockSpec((D, D), full),
            pl.BlockSpec((1, D), full),
            pl.BlockSpec((D, D), full),
            pl.BlockSpec((1, D), full),
            pl.BlockSpec((D, 3), full),
            pl.BlockSpec((1, 3), full),
        ],
        out_specs=pl.BlockSpec((BN, 3), blk),
        out_shape=jax.ShapeDtypeStruct((N, 3), jnp.float32),
    )(h, a0, a1, a2, a3, w1a, w1b, b1, w2, b2, wd1, bd1, wd2, bd2)


def _decode_body(h_ref, w1_ref, b1_ref, w2_ref, b2_ref, o_ref):
    h = h_ref[...]
    d = _silu(jnp.dot(h, w1_ref[...], preferred_element_type=jnp.float32)
              + b1_ref[...])
    o_ref[...] = (jnp.dot(d, w2_ref[...], preferred_element_type=jnp.float32)
                  + b2_ref[...])


def _decode(h, w1, b1, w2, b2):
    grid = (N // BN,)
    full = lambda i: (0, 0)
    return pl.pallas_call(
        _decode_body,
        grid=grid,
        in_specs=[
            pl.BlockSpec((BN, D), lambda i: (i, 0)),
            pl.BlockSpec((D, D), full),
            pl.BlockSpec((1, D), full),
            pl.BlockSpec((D, 3), full),
            pl.BlockSpec((1, 3), full),
        ],
        out_specs=pl.BlockSpec((BN, 3), lambda i: (i, 0)),
        out_shape=jax.ShapeDtypeStruct((N, 3), jnp.float32),
    )(h, w1, b1, w2, b2)


def kernel(nodes, loc, edges, vel, edge_attr, _, W_emb, b_emb, We1, be1,
           We2, be2, Wn1, bn1, Wn2, bn2, Wd1, bd1, Wd2, bd2):
    row = edges[0]
    col = edges[1]
    rowp = [_pad_worker_idx(row[:E2]), _pad_worker_idx(row[E2:])]
    colp = [_pad_worker_idx(col[:E2]), _pad_worker_idx(col[E2:])]
    ea = [edge_attr[:E2], edge_attr[E2:]]
    zeros = jnp.zeros((NP, D), jnp.float32)

    def _wpq(i):
        A = We1[i, :D, :]
        B = We1[i, D:2 * D, :]
        return jnp.concatenate([A[:, 0::2], A[:, 1::2],
                                B[:, 0::2], B[:, 1::2]], axis=1)

    wpq = [_wpq(i) for i in range(L)]

    h, p, q = _embed(loc, vel, W_emb[:3], W_emb[3:], b_emb.reshape(1, D),
                     wpq[0])
    permi = jnp.concatenate([jnp.arange(0, D, 2), jnp.arange(1, D, 2)])
    for i in range(L):
        c2 = We1[i, 2 * D:, :][:, permi]
        b1 = be1[i][permi].reshape(1, D)
        w2 = We2[i][permi, :]
        b2 = be2[i].reshape(1, D)
        z0 = _gather_add(p, q, rowp[0], colp[0])
        m0 = _edge_mlp(z0, ea[0], c2, b1, w2, b2)
        z1 = _gather_add(p, q, rowp[1], colp[1])
        agg0 = _scatter_add(m0, rowp[0], zeros)
        m1 = _edge_mlp(z1, ea[1], c2, b1, w2, b2)
        agg1 = _scatter_add(m1, rowp[1], zeros)
        if i < L - 1:
            h, p, q = _node_update(h, agg0[:N], agg0[NP:NP + N],
                                   agg1[:N], agg1[NP:NP + N],
                                   Wn1[i, :D, :], Wn1[i, D:, :],
                                   bn1[i].reshape(1, D), Wn2[i],
                                   bn2[i].reshape(1, D), wpq[i + 1])
        else:
            return _node_decode(h, agg0[:N], agg0[NP:NP + N],
                                agg1[:N], agg1[NP:NP + N],
                                Wn1[i, :D, :], Wn1[i, D:, :],
                                bn1[i].reshape(1, D), Wn2[i],
                                bn2[i].reshape(1, D),
                                Wd1, bd1.reshape(1, D), Wd2,
                                bd2.reshape(1, 3))


# speedup vs baseline: 1.0942x; 1.0145x over previous
import functools

import jax
import jax.numpy as jnp
from jax import lax
from jax.experimental import pallas as pl
from jax.experimental.pallas import tpu as pltpu
from jax.experimental.pallas import tpu_sc as plsc

N = 10000
E = 320000
D = 128
L = 4

NC = 2
NS = 16
NW = NC * NS
CH = 128
CHUNKS = E // CH
E2 = E // 2
HCHUNKS = E2 // CH
ROWS_PER_TILE = 632
NP = ROWS_PER_TILE * NS

@functools.lru_cache(maxsize=None)
def _sc_mesh():
    return plsc.VectorSubcoreMesh(
        core_axis_name="c", subcore_axis_name="s",
        num_cores=NC, num_subcores=NS)


def _silu(v):
    return v * (1.0 / (1.0 + jnp.exp(-v)))


def _rne_bf16_bits(x):
    rb = jax.lax.bitcast_convert_type(x, jnp.uint32)
    return (rb + jnp.uint32(0x7FFF) + ((rb >> 16) & jnp.uint32(1))) >> 16


def _pack_bf16_pair(lo, hi):
    w = _rne_bf16_bits(lo) | (_rne_bf16_bits(hi) << 16)
    return jax.lax.bitcast_convert_type(w, jnp.int32)


def _unpack_bf16_pair(w):
    u = jax.lax.bitcast_convert_type(w, jnp.uint32)
    lo = jax.lax.bitcast_convert_type(u << 16, jnp.float32)
    hi = jax.lax.bitcast_convert_type(u & jnp.uint32(0xFFFF0000), jnp.float32)
    return lo, hi


MAXC = 40


def _pad_worker_idx(idx):
    idx2d = idx.reshape(HCHUNKS, CH)
    per = HCHUNKS // NW
    rem = HCHUNKS % NW
    slabs = []
    for w in range(NW):
        b = w * per + min(w, rem)
        cnt = per + (1 if w < rem else 0)
        slabs.append(jnp.pad(idx2d[b:b + cnt], ((0, MAXC - cnt), (0, 0))))
    return jnp.stack(slabs)


def _worker_split(wid):
    per = HCHUNKS // NW
    rem = HCHUNKS % NW
    base = wid * per + jnp.minimum(wid, rem)
    cnt = per + jnp.where(wid < rem, 1, 0)
    return base, cnt


NBUF = 3


def _gather_add_body(p_hbm, q_hbm, row_hbm, col_hbm, z_hbm,
                     ridx, cidx, pbuf0, qbuf0, pbuf1, qbuf1, pbuf2, qbuf2,
                     isem, gsem0, gsem1, gsem2, wsem0, wsem1, wsem2):
    c = lax.axis_index("c")
    s = lax.axis_index("s")
    wid = s * NC + c
    base, cnt = _worker_split(wid)

    pltpu.async_copy(row_hbm.at[wid], ridx, isem)
    pltpu.async_copy(col_hbm.at[wid], cidx, isem)
    pltpu.make_async_copy(row_hbm.at[wid], ridx, isem).wait()
    pltpu.make_async_copy(col_hbm.at[wid], cidx, isem).wait()

    pbufs = (pbuf0, pbuf1, pbuf2)
    qbufs = (qbuf0, qbuf1, qbuf2)
    gsems = (gsem0, gsem1, gsem2)
    wsems = (wsem0, wsem1, wsem2)

    def _issue(j, b):
        pltpu.async_copy(p_hbm.at[ridx.at[j]], pbufs[b], gsems[b])
        pltpu.async_copy(q_hbm.at[cidx.at[j]], qbufs[b], gsems[b])

    def _process(j, b):
        pltpu.make_async_copy(p_hbm.at[ridx.at[j]], pbufs[b], gsems[b]).wait()
        pltpu.make_async_copy(q_hbm.at[cidx.at[j]], qbufs[b], gsems[b]).wait()
        pb, qb = pbufs[b], qbufs[b]

        def add_rows(i, carry):
            r = i * 4
            for rr in range(4):
                for cc in range(D // 16):
                    sl = pl.ds(cc * 16, 16)
                    pb[r + rr, sl] = pb[r + rr, sl] + qb[r + rr, sl]
            return carry

        lax.fori_loop(0, CH // 4, add_rows, 0)
        pltpu.async_copy(pb, z_hbm.at[pl.ds((base + j) * CH, CH)], wsems[b])

    def step(i, carry):
        for b in range(NBUF):
            j = i * NBUF + b

            @pl.when(j < cnt)
            def _():
                @pl.when(j >= NBUF)
                def _():
                    pltpu.make_async_copy(
                        pbufs[b], z_hbm.at[pl.ds(0, CH)], wsems[b]).wait()

                _issue(j, b)

            @pl.when((j >= 2) & (j <= cnt + 1))
            def _():
                _process(j - 2, (b + 1) % NBUF)
        return carry

    lax.fori_loop(0, (cnt + NBUF) // NBUF, step, 0)
    pltpu.make_async_copy(pbuf0, z_hbm.at[pl.ds(0, CH)], wsem0).wait()
    pltpu.make_async_copy(pbuf1, z_hbm.at[pl.ds(0, CH)], wsem1).wait()
    pltpu.make_async_copy(pbuf2, z_hbm.at[pl.ds(0, CH)], wsem2).wait()


@functools.lru_cache(maxsize=None)
def _gather_add_kernel():
    return pl.kernel(
        _gather_add_body,
        out_type=jax.ShapeDtypeStruct((E2, D), jnp.float32),
        mesh=_sc_mesh(),
        scratch_types=[
            pltpu.VMEM((MAXC, CH), jnp.int32),
            pltpu.VMEM((MAXC, CH), jnp.int32),
            pltpu.VMEM((CH, D), jnp.float32),
            pltpu.VMEM((CH, D), jnp.float32),
            pltpu.VMEM((CH, D), jnp.float32),
            pltpu.VMEM((CH, D), jnp.float32),
            pltpu.VMEM((CH, D), jnp.float32),
            pltpu.VMEM((CH, D), jnp.float32),
            pltpu.SemaphoreType.DMA,
            pltpu.SemaphoreType.DMA,
            pltpu.SemaphoreType.DMA,
            pltpu.SemaphoreType.DMA,
            pltpu.SemaphoreType.DMA,
            pltpu.SemaphoreType.DMA,
            pltpu.SemaphoreType.DMA,
        ],
    )


def _gather_add(p, q, rowp, colp):
    return _gather_add_kernel()(p, q, rowp, colp)


def _scatter_add_body(m_hbm, row_hbm, zeros_hbm, agg_hbm,
                      ridx, mbuf0, mbuf1, accum, lsem0, lsem1):
    c = lax.axis_index("c")
    s = lax.axis_index("s")
    wid = s * NC + c
    base, cnt = _worker_split(wid)
    zsrc = zeros_hbm.at[pl.ds(s * ROWS_PER_TILE, ROWS_PER_TILE)]
    zdst = accum.at[pl.ds(s * ROWS_PER_TILE, ROWS_PER_TILE)]
    pltpu.async_copy(zsrc, zdst, lsem0)
    pltpu.async_copy(row_hbm.at[wid], ridx, lsem1)
    pltpu.make_async_copy(zsrc, zdst, lsem0).wait()
    pltpu.make_async_copy(row_hbm.at[wid], ridx, lsem1).wait()
    plsc.subcore_barrier()

    mbufs = (mbuf0, mbuf1)
    lsems = (lsem0, lsem1)

    def step(i, carry):
        for b in range(2):
            j = i * 2 + b

            @pl.when(j < cnt)
            def _():
                pltpu.async_copy(m_hbm.at[pl.ds((base + j) * CH, CH)],
                                 mbufs[b], lsems[b])

            @pl.when((j >= 1) & (j <= cnt))
            def _():
                pltpu.make_async_copy(
                    m_hbm.at[pl.ds(base * CH, CH)],
                    mbufs[1 - b], lsems[1 - b]).wait()
                pltpu.sync_copy(mbufs[1 - b], accum.at[ridx.at[j - 1]],
                                add=True)
        return carry

    lax.fori_loop(0, (cnt + 2) // 2, step, 0)
    plsc.subcore_barrier()
    r0 = s * ROWS_PER_TILE
    pltpu.sync_copy(accum.at[pl.ds(r0, ROWS_PER_TILE)],
                    agg_hbm.at[pl.ds(c * NP + r0, ROWS_PER_TILE)])


@functools.lru_cache(maxsize=None)
def _scatter_add_kernel():
    return pl.kernel(
        _scatter_add_body,
        out_type=jax.ShapeDtypeStruct((NC * NP, D), jnp.float32),
        mesh=_sc_mesh(),
        scratch_types=[
            pltpu.VMEM((MAXC, CH), jnp.int32),
            pltpu.VMEM((CH, D), jnp.float32),
            pltpu.VMEM((CH, D), jnp.float32),
            pltpu.VMEM_SHARED((NP, D), jnp.float32),
            pltpu.SemaphoreType.DMA,
            pltpu.SemaphoreType.DMA,
        ],
    )


def _scatter_add(m, rowp, zeros):
    return _scatter_add_kernel()(m, rowp, zeros)


BE = 8000
BN = 2000


def _edge_mlp_body(z_ref, ea_ref, c_ref, be1_ref, w2_ref, be2_ref, m_ref):
    z = z_ref[...]
    ea = ea_ref[...]
    z = z + ea[:, 0:1] * c_ref[0:1, :] + ea[:, 1:2] * c_ref[1:2, :] + be1_ref[...]
    a = _silu(z)
    m = jnp.dot(a, w2_ref[...], preferred_element_type=jnp.float32) + be2_ref[...]
    m_ref[...] = _silu(m)


def _edge_mlp(z, ea, c2, be1, w2, be2):
    grid = (E2 // BE,)
    return pl.pallas_call(
        _edge_mlp_body,
        grid=grid,
        in_specs=[
            pl.BlockSpec((BE, D), lambda i: (i, 0)),
            pl.BlockSpec((BE, 2), lambda i: (i, 0)),
            pl.BlockSpec((2, D), lambda i: (0, 0)),
            pl.BlockSpec((1, D), lambda i: (0, 0)),
            pl.BlockSpec((D, D), lambda i: (0, 0)),
            pl.BlockSpec((1, D), lambda i: (0, 0)),
        ],
        out_specs=pl.BlockSpec((BE, D), lambda i: (i, 0)),
        out_shape=jax.ShapeDtypeStruct((E2, D), jnp.float32),
    )(z, ea, c2, be1, w2, be2)


def _node_body(h_ref, a0_ref, a1_ref, a2_ref, a3_ref, w1a_ref, w1b_ref,
               b1_ref, w2_ref, b2_ref, wpq_ref, h_out, p_out, q_out):
    h = h_ref[...]
    agg = (a0_ref[...] + a1_ref[...]) + (a2_ref[...] + a3_ref[...])
    u = (jnp.dot(h, w1a_ref[...], preferred_element_type=jnp.float32)
         + jnp.dot(agg, w1b_ref[...], preferred_element_type=jnp.float32)
         + b1_ref[...])
    u = _silu(u)
    hn = h + jnp.dot(u, w2_ref[...], preferred_element_type=jnp.float32) + b2_ref[...]
    h_out[...] = hn
    pq = jnp.dot(hn, wpq_ref[...], preferred_element_type=jnp.float32)
    p_out[...] = pq[:, :D]
    q_out[...] = pq[:, D:]


def _node_update(h, a0, a1, a2, a3, w1a, w1b, b1, w2, b2, wpq):
    grid = (N // BN,)
    full = lambda i: (0, 0)
    return pl.pallas_call(
        _node_body,
        grid=grid,
        in_specs=[
            pl.BlockSpec((BN, D), lambda i: (i, 0)),
            pl.BlockSpec((BN, D), lambda i: (i, 0)),
            pl.BlockSpec((BN, D), lambda i: (i, 0)),
            pl.BlockSpec((BN, D), lambda i: (i, 0)),
            pl.BlockSpec((BN, D), lambda i: (i, 0)),
            pl.BlockSpec((D, D), full),
            pl.BlockSpec((D, D), full),
            pl.BlockSpec((1, D), full),
            pl.BlockSpec((D, D), full),
            pl.BlockSpec((1, D), full),
            pl.BlockSpec((D, 2 * D), full),
        ],
        out_specs=[
            pl.BlockSpec((BN, D), lambda i: (i, 0)),
            pl.BlockSpec((BN, D), lambda i: (i, 0)),
            pl.BlockSpec((BN, D), lambda i: (i, 0)),
        ],
        out_shape=[
            jax.ShapeDtypeStruct((N, D), jnp.float32),
            jax.ShapeDtypeStruct((N, D), jnp.float32),
            jax.ShapeDtypeStruct((N, D), jnp.float32),
        ],
    )(h, a0, a1, a2, a3, w1a, w1b, b1, w2, b2, wpq)


def _embed_body(loc_ref, vel_ref, wl_ref, wv_ref, b_ref, wpq_ref,
                h_out, p_out, q_out):
    loc = loc_ref[...]
    vel = vel_ref[...]
    h = b_ref[...] + jnp.zeros((loc.shape[0], D), jnp.float32)
    for j in range(3):
        h = h + loc[:, j:j + 1] * wl_ref[j:j + 1, :]
        h = h + vel[:, j:j + 1] * wv_ref[j:j + 1, :]
    h_out[...] = h
    pq = jnp.dot(h, wpq_ref[...], preferred_element_type=jnp.float32)
    p_out[...] = pq[:, :D]
    q_out[...] = pq[:, D:]


def _embed(loc, vel, wl, wv, b, wpq):
    grid = (N // BN,)
    full = lambda i: (0, 0)
    return pl.pallas_call(
        _embed_body,
        grid=grid,
        in_specs=[
            pl.BlockSpec((BN, 3), lambda i: (i, 0)),
            pl.BlockSpec((BN, 3), lambda i: (i, 0)),
            pl.BlockSpec((3, D), full),
            pl.BlockSpec((3, D), full),
            pl.BlockSpec((1, D), full),
            pl.BlockSpec((D, 2 * D), full),
        ],
        out_specs=[
            pl.BlockSpec((BN, D), lambda i: (i, 0)),
            pl.BlockSpec((BN, D), lambda i: (i, 0)),
            pl.BlockSpec((BN, D), lambda i: (i, 0)),
        ],
        out_shape=[
            jax.ShapeDtypeStruct((N, D), jnp.float32),
            jax.ShapeDtypeStruct((N, D), jnp.float32),
            jax.ShapeDtypeStruct((N, D), jnp.float32),
        ],
    )(loc, vel, wl, wv, b, wpq)


def _node_decode_body(h_ref, a0_ref, a1_ref, a2_ref, a3_ref, w1a_ref,
                      w1b_ref, b1_ref, w2_ref, b2_ref, wd1_ref, bd1_ref,
                      wd2_ref, bd2_ref, o_ref):
    h = h_ref[...]
    agg = (a0_ref[...] + a1_ref[...]) + (a2_ref[...] + a3_ref[...])
    u = (jnp.dot(h, w1a_ref[...], preferred_element_type=jnp.float32)
         + jnp.dot(agg, w1b_ref[...], preferred_element_type=jnp.float32)
         + b1_ref[...])
    u = _silu(u)
    hn = h + jnp.dot(u, w2_ref[...], preferred_element_type=jnp.float32) + b2_ref[...]
    d = _silu(jnp.dot(hn, wd1_ref[...], preferred_element_type=jnp.float32)
              + bd1_ref[...])
    o_ref[...] = (jnp.dot(d, wd2_ref[...], preferred_element_type=jnp.float32)
                  + bd2_ref[...])


def _node_decode(h, a0, a1, a2, a3, w1a, w1b, b1, w2, b2, wd1, bd1, wd2, bd2):
    grid = (N // BN,)
    full = lambda i: (0, 0)
    blk = lambda i: (i, 0)
    return pl.pallas_call(
        _node_decode_body,
        grid=grid,
        in_specs=[
            pl.BlockSpec((BN, D), blk),
            pl.BlockSpec((BN, D), blk),
            pl.BlockSpec((BN, D), blk),
            pl.BlockSpec((BN, D), blk),
            pl.BlockSpec((BN, D), blk),
            pl.BlockSpec((D, D), full),
            pl.BlockSpec((D, D), full),
            pl.BlockSpec((1, D), full),
            pl.BlockSpec((D, D), full),
            pl.BlockSpec((1, D), full),
            pl.BlockSpec((D, D), full),
            pl.BlockSpec((1, D), full),
            pl.BlockSpec((D, 3), full),
            pl.BlockSpec((1, 3), full),
        ],
        out_specs=pl.BlockSpec((BN, 3), blk),
        out_shape=jax.ShapeDtypeStruct((N, 3), jnp.float32),
    )(h, a0, a1, a2, a3, w1a, w1b, b1, w2, b2, wd1, bd1, wd2, bd2)


def _decode_body(h_ref, w1_ref, b1_ref, w2_ref, b2_ref, o_ref):
    h = h_ref[...]
    d = _silu(jnp.dot(h, w1_ref[...], preferred_element_type=jnp.float32)
              + b1_ref[...])
    o_ref[...] = (jnp.dot(d, w2_ref[...], preferred_element_type=jnp.float32)
                  + b2_ref[...])


def _decode(h, w1, b1, w2, b2):
    grid = (N // BN,)
    full = lambda i: (0, 0)
    return pl.pallas_call(
        _decode_body,
        grid=grid,
        in_specs=[
            pl.BlockSpec((BN, D), lambda i: (i, 0)),
            pl.BlockSpec((D, D), full),
            pl.BlockSpec((1, D), full),
            pl.BlockSpec((D, 3), full),
            pl.BlockSpec((1, 3), full),
        ],
        out_specs=pl.BlockSpec((BN, 3), lambda i: (i, 0)),
        out_shape=jax.ShapeDtypeStruct((N, 3), jnp.float32),
    )(h, w1, b1, w2, b2)


def kernel(nodes, loc, edges, vel, edge_attr, _, W_emb, b_emb, We1, be1,
           We2, be2, Wn1, bn1, Wn2, bn2, Wd1, bd1, Wd2, bd2):
    row = edges[0]
    col = edges[1]
    rowp = [_pad_worker_idx(row[:E2]), _pad_worker_idx(row[E2:])]
    colp = [_pad_worker_idx(col[:E2]), _pad_worker_idx(col[E2:])]
    ea = [edge_attr[:E2], edge_attr[E2:]]
    zeros = jnp.zeros((NP, D), jnp.float32)

    def _wpq(i):
        A = We1[i, :D, :]
        B = We1[i, D:2 * D, :]
        return jnp.concatenate([A[:, 0::2], A[:, 1::2],
                                B[:, 0::2], B[:, 1::2]], axis=1)

    wpq = [_wpq(i) for i in range(L)]

    h, p, q = _embed(loc, vel, W_emb[:3], W_emb[3:], b_emb.reshape(1, D),
                     wpq[0])
    permi = jnp.concatenate([jnp.arange(0, D, 2), jnp.arange(1, D, 2)])
    for i in range(L):
        c2 = We1[i, 2 * D:, :][:, permi]
        b1 = be1[i][permi].reshape(1, D)
        w2 = We2[i][permi, :]
        b2 = be2[i].reshape(1, D)
        z0 = _gather_add(p, q, rowp[0], colp[0])
        m0 = _edge_mlp(z0, ea[0], c2, b1, w2, b2)
        z1 = _gather_add(p, q, rowp[1], colp[1])
        agg0 = _scatter_add(m0, rowp[0], zeros)
        m1 = _edge_mlp(z1, ea[1], c2, b1, w2, b2)
        agg1 = _scatter_add(m1, rowp[1], zeros)
        if i < L - 1:
            h, p, q = _node_update(h, agg0[:N], agg0[NP:NP + N],
                                   agg1[:N], agg1[NP:NP + N],
                                   Wn1[i, :D, :], Wn1[i, D:, :],
                                   bn1[i].reshape(1, D), Wn2[i],
                                   bn2[i].reshape(1, D), wpq[i + 1])
        else:
            return _node_decode(h, agg0[:N], agg0[NP:NP + N],
                                agg1[:N], agg1[NP:NP + N],
                                Wn1[i, :D, :], Wn1[i, D:, :],
                                bn1[i].reshape(1, D), Wn2[i],
                                bn2[i].reshape(1, D),
                                Wd1, bd1.reshape(1, D), Wd2,
                                bd2.reshape(1, 3))


# speedup vs baseline: 1.0945x; 1.0002x over previous
import functools

import jax
import jax.numpy as jnp
from jax import lax
from jax.experimental import pallas as pl
from jax.experimental.pallas import tpu as pltpu
from jax.experimental.pallas import tpu_sc as plsc

N = 10000
E = 320000
D = 128
L = 4

NC = 2
NS = 16
NW = NC * NS
CH = 128
CHUNKS = E // CH
E2 = E // 2
HCHUNKS = E2 // CH
ROWS_PER_TILE = 632
NP = ROWS_PER_TILE * NS

@functools.lru_cache(maxsize=None)
def _sc_mesh():
    return plsc.VectorSubcoreMesh(
        core_axis_name="c", subcore_axis_name="s",
        num_cores=NC, num_subcores=NS)


def _silu(v):
    return v * (1.0 / (1.0 + jnp.exp(-v)))


MAXC = 40


def _pad_worker_idx(idx):
    idx2d = idx.reshape(HCHUNKS, CH)
    per = HCHUNKS // NW
    rem = HCHUNKS % NW
    slabs = []
    for w in range(NW):
        b = w * per + min(w, rem)
        cnt = per + (1 if w < rem else 0)
        slabs.append(jnp.pad(idx2d[b:b + cnt], ((0, MAXC - cnt), (0, 0))))
    return jnp.stack(slabs)


def _worker_split(wid):
    per = HCHUNKS // NW
    rem = HCHUNKS % NW
    base = wid * per + jnp.minimum(wid, rem)
    cnt = per + jnp.where(wid < rem, 1, 0)
    return base, cnt


NBUF = 3


def _gather_add_body(p_hbm, q_hbm, row_hbm, col_hbm, z_hbm,
                     ridx, cidx, pbuf0, qbuf0, pbuf1, qbuf1, pbuf2, qbuf2,
                     isem, gsem0, gsem1, gsem2, wsem0, wsem1, wsem2):
    c = lax.axis_index("c")
    s = lax.axis_index("s")
    wid = s * NC + c
    base, cnt = _worker_split(wid)

    pltpu.async_copy(row_hbm.at[wid], ridx, isem)
    pltpu.async_copy(col_hbm.at[wid], cidx, isem)
    pltpu.make_async_copy(row_hbm.at[wid], ridx, isem).wait()
    pltpu.make_async_copy(col_hbm.at[wid], cidx, isem).wait()

    pbufs = (pbuf0, pbuf1, pbuf2)
    qbufs = (qbuf0, qbuf1, qbuf2)
    gsems = (gsem0, gsem1, gsem2)
    wsems = (wsem0, wsem1, wsem2)

    def _issue(j, b):
        pltpu.async_copy(p_hbm.at[ridx.at[j]], pbufs[b], gsems[b])
        pltpu.async_copy(q_hbm.at[cidx.at[j]], qbufs[b], gsems[b])

    def _process(j, b):
        pltpu.make_async_copy(p_hbm.at[ridx.at[j]], pbufs[b], gsems[b]).wait()
        pltpu.make_async_copy(q_hbm.at[cidx.at[j]], qbufs[b], gsems[b]).wait()
        pb, qb = pbufs[b], qbufs[b]

        def add_rows(i, carry):
            r = i * 4
            for rr in range(4):
                for cc in range(D // 16):
                    sl = pl.ds(cc * 16, 16)
                    pb[r + rr, sl] = pb[r + rr, sl] + qb[r + rr, sl]
            return carry

        lax.fori_loop(0, CH // 4, add_rows, 0)
        pltpu.async_copy(pb, z_hbm.at[pl.ds((base + j) * CH, CH)], wsems[b])

    def step(i, carry):
        for b in range(NBUF):
            j = i * NBUF + b

            @pl.when(j < cnt)
            def _():
                @pl.when(j >= NBUF)
                def _():
                    pltpu.make_async_copy(
                        pbufs[b], z_hbm.at[pl.ds(0, CH)], wsems[b]).wait()

                _issue(j, b)

            @pl.when((j >= 2) & (j <= cnt + 1))
            def _():
                _process(j - 2, (b + 1) % NBUF)
        return carry

    lax.fori_loop(0, (cnt + NBUF) // NBUF, step, 0)
    pltpu.make_async_copy(pbuf0, z_hbm.at[pl.ds(0, CH)], wsem0).wait()
    pltpu.make_async_copy(pbuf1, z_hbm.at[pl.ds(0, CH)], wsem1).wait()
    pltpu.make_async_copy(pbuf2, z_hbm.at[pl.ds(0, CH)], wsem2).wait()


@functools.lru_cache(maxsize=None)
def _gather_add_kernel():
    return pl.kernel(
        _gather_add_body,
        out_type=jax.ShapeDtypeStruct((E2, D), jnp.float32),
        mesh=_sc_mesh(),
        scratch_types=[
            pltpu.VMEM((MAXC, CH), jnp.int32),
            pltpu.VMEM((MAXC, CH), jnp.int32),
            pltpu.VMEM((CH, D), jnp.float32),
            pltpu.VMEM((CH, D), jnp.float32),
            pltpu.VMEM((CH, D), jnp.float32),
            pltpu.VMEM((CH, D), jnp.float32),
            pltpu.VMEM((CH, D), jnp.float32),
            pltpu.VMEM((CH, D), jnp.float32),
            pltpu.SemaphoreType.DMA,
            pltpu.SemaphoreType.DMA,
            pltpu.SemaphoreType.DMA,
            pltpu.SemaphoreType.DMA,
            pltpu.SemaphoreType.DMA,
            pltpu.SemaphoreType.DMA,
            pltpu.SemaphoreType.DMA,
        ],
    )


def _gather_add(p, q, rowp, colp):
    return _gather_add_kernel()(p, q, rowp, colp)


def _scatter_add_body(m_hbm, row_hbm, zeros_hbm, agg_hbm,
                      ridx, mbuf0, mbuf1, accum, lsem0, lsem1):
    c = lax.axis_index("c")
    s = lax.axis_index("s")
    wid = s * NC + c
    base, cnt = _worker_split(wid)
    zsrc = zeros_hbm.at[pl.ds(s * ROWS_PER_TILE, ROWS_PER_TILE)]
    zdst = accum.at[pl.ds(s * ROWS_PER_TILE, ROWS_PER_TILE)]
    pltpu.async_copy(zsrc, zdst, lsem0)
    pltpu.async_copy(row_hbm.at[wid], ridx, lsem1)
    pltpu.make_async_copy(zsrc, zdst, lsem0).wait()
    pltpu.make_async_copy(row_hbm.at[wid], ridx, lsem1).wait()
    plsc.subcore_barrier()

    mbufs = (mbuf0, mbuf1)
    lsems = (lsem0, lsem1)

    def step(i, carry):
        for b in range(2):
            j = i * 2 + b

            @pl.when(j < cnt)
            def _():
                pltpu.async_copy(m_hbm.at[pl.ds((base + j) * CH, CH)],
                                 mbufs[b], lsems[b])

            @pl.when((j >= 1) & (j <= cnt))
            def _():
                pltpu.make_async_copy(
                    m_hbm.at[pl.ds(base * CH, CH)],
                    mbufs[1 - b], lsems[1 - b]).wait()
                pltpu.sync_copy(mbufs[1 - b], accum.at[ridx.at[j - 1]],
                                add=True)
        return carry

    lax.fori_loop(0, (cnt + 2) // 2, step, 0)
    plsc.subcore_barrier()
    r0 = s * ROWS_PER_TILE
    pltpu.sync_copy(accum.at[pl.ds(r0, ROWS_PER_TILE)],
                    agg_hbm.at[pl.ds(c * NP + r0, ROWS_PER_TILE)])


@functools.lru_cache(maxsize=None)
def _scatter_add_kernel():
    return pl.kernel(
        _scatter_add_body,
        out_type=jax.ShapeDtypeStruct((NC * NP, D), jnp.float32),
        mesh=_sc_mesh(),
        scratch_types=[
            pltpu.VMEM((MAXC, CH), jnp.int32),
            pltpu.VMEM((CH, D), jnp.float32),
            pltpu.VMEM((CH, D), jnp.float32),
            pltpu.VMEM_SHARED((NP, D), jnp.float32),
            pltpu.SemaphoreType.DMA,
            pltpu.SemaphoreType.DMA,
        ],
    )


def _scatter_add(m, rowp, zeros):
    return _scatter_add_kernel()(m, rowp, zeros)


BE = 8000
BN = 2000


def _edge_mlp_body(z_ref, ea_ref, c_ref, be1_ref, w2_ref, be2_ref, m_ref):
    z = z_ref[...]
    ea = ea_ref[...]
    z = z + ea[:, 0:1] * c_ref[0:1, :] + ea[:, 1:2] * c_ref[1:2, :] + be1_ref[...]
    a = _silu(z)
    m = jnp.dot(a, w2_ref[...], preferred_element_type=jnp.float32) + be2_ref[...]
    m_ref[...] = _silu(m)


def _edge_mlp(z, ea, c2, be1, w2, be2):
    grid = (E2 // BE,)
    return pl.pallas_call(
        _edge_mlp_body,
        grid=grid,
        in_specs=[
            pl.BlockSpec((BE, D), lambda i: (i, 0)),
            pl.BlockSpec((BE, 2), lambda i: (i, 0)),
            pl.BlockSpec((2, D), lambda i: (0, 0)),
            pl.BlockSpec((1, D), lambda i: (0, 0)),
            pl.BlockSpec((D, D), lambda i: (0, 0)),
            pl.BlockSpec((1, D), lambda i: (0, 0)),
        ],
        out_specs=pl.BlockSpec((BE, D), lambda i: (i, 0)),
        out_shape=jax.ShapeDtypeStruct((E2, D), jnp.float32),
    )(z, ea, c2, be1, w2, be2)


def _node_body(h_ref, a0_ref, a1_ref, a2_ref, a3_ref, w1a_ref, w1b_ref,
               b1_ref, w2_ref, b2_ref, wpq_ref, h_out, p_out, q_out):
    h = h_ref[...]
    agg = (a0_ref[...] + a1_ref[...]) + (a2_ref[...] + a3_ref[...])
    u = (jnp.dot(h, w1a_ref[...], preferred_element_type=jnp.float32)
         + jnp.dot(agg, w1b_ref[...], preferred_element_type=jnp.float32)
         + b1_ref[...])
    u = _silu(u)
    hn = h + jnp.dot(u, w2_ref[...], preferred_element_type=jnp.float32) + b2_ref[...]
    h_out[...] = hn
    pq = jnp.dot(hn, wpq_ref[...], preferred_element_type=jnp.float32)
    p_out[...] = pq[:, :D]
    q_out[...] = pq[:, D:]


def _node_update(h, a0, a1, a2, a3, w1a, w1b, b1, w2, b2, wpq):
    grid = (N // BN,)
    full = lambda i: (0, 0)
    return pl.pallas_call(
        _node_body,
        grid=grid,
        in_specs=[
            pl.BlockSpec((BN, D), lambda i: (i, 0)),
            pl.BlockSpec((BN, D), lambda i: (i, 0)),
            pl.BlockSpec((BN, D), lambda i: (i, 0)),
            pl.BlockSpec((BN, D), lambda i: (i, 0)),
            pl.BlockSpec((BN, D), lambda i: (i, 0)),
            pl.BlockSpec((D, D), full),
            pl.BlockSpec((D, D), full),
            pl.BlockSpec((1, D), full),
            pl.BlockSpec((D, D), full),
            pl.BlockSpec((1, D), full),
            pl.BlockSpec((D, 2 * D), full),
        ],
        out_specs=[
            pl.BlockSpec((BN, D), lambda i: (i, 0)),
            pl.BlockSpec((BN, D), lambda i: (i, 0)),
            pl.BlockSpec((BN, D), lambda i: (i, 0)),
        ],
        out_shape=[
            jax.ShapeDtypeStruct((N, D), jnp.float32),
            jax.ShapeDtypeStruct((N, D), jnp.float32),
            jax.ShapeDtypeStruct((N, D), jnp.float32),
        ],
    )(h, a0, a1, a2, a3, w1a, w1b, b1, w2, b2, wpq)


def _embed_body(loc_ref, vel_ref, wl_ref, wv_ref, b_ref, wpq_ref,
                h_out, p_out, q_out):
    loc = loc_ref[...]
    vel = vel_ref[...]
    h = b_ref[...] + jnp.zeros((loc.shape[0], D), jnp.float32)
    for j in range(3):
        h = h + loc[:, j:j + 1] * wl_ref[j:j + 1, :]
        h = h + vel[:, j:j + 1] * wv_ref[j:j + 1, :]
    h_out[...] = h
    pq = jnp.dot(h, wpq_ref[...], preferred_element_type=jnp.float32)
    p_out[...] = pq[:, :D]
    q_out[...] = pq[:, D:]


def _embed(loc, vel, wl, wv, b, wpq):
    grid = (N // BN,)
    full = lambda i: (0, 0)
    return pl.pallas_call(
        _embed_body,
        grid=grid,
        in_specs=[
            pl.BlockSpec((BN, 3), lambda i: (i, 0)),
            pl.BlockSpec((BN, 3), lambda i: (i, 0)),
            pl.BlockSpec((3, D), full),
            pl.BlockSpec((3, D), full),
            pl.BlockSpec((1, D), full),
            pl.BlockSpec((D, 2 * D), full),
        ],
        out_specs=[
            pl.BlockSpec((BN, D), lambda i: (i, 0)),
            pl.BlockSpec((BN, D), lambda i: (i, 0)),
            pl.BlockSpec((BN, D), lambda i: (i, 0)),
        ],
        out_shape=[
            jax.ShapeDtypeStruct((N, D), jnp.float32),
            jax.ShapeDtypeStruct((N, D), jnp.float32),
            jax.ShapeDtypeStruct((N, D), jnp.float32),
        ],
    )(loc, vel, wl, wv, b, wpq)


def _node_decode_body(h_ref, a0_ref, a1_ref, a2_ref, a3_ref, w1a_ref,
                      w1b_ref, b1_ref, w2_ref, b2_ref, wd1_ref, bd1_ref,
                      wd2_ref, bd2_ref, o_ref):
    h = h_ref[...]
    agg = (a0_ref[...] + a1_ref[...]) + (a2_ref[...] + a3_ref[...])
    u = (jnp.dot(h, w1a_ref[...], preferred_element_type=jnp.float32)
         + jnp.dot(agg, w1b_ref[...], preferred_element_type=jnp.float32)
         + b1_ref[...])
    u = _silu(u)
    hn = h + jnp.dot(u, w2_ref[...], preferred_element_type=jnp.float32) + b2_ref[...]
    d = _silu(jnp.dot(hn, wd1_ref[...], preferred_element_type=jnp.float32)
              + bd1_ref[...])
    o_ref[...] = (jnp.dot(d, wd2_ref[...], preferred_element_type=jnp.float32)
                  + bd2_ref[...])


def _node_decode(h, a0, a1, a2, a3, w1a, w1b, b1, w2, b2, wd1, bd1, wd2, bd2):
    grid = (N // BN,)
    full = lambda i: (0, 0)
    blk = lambda i: (i, 0)
    return pl.pallas_call(
        _node_decode_body,
        grid=grid,
        in_specs=[
            pl.BlockSpec((BN, D), blk),
            pl.BlockSpec((BN, D), blk),
            pl.BlockSpec((BN, D), blk),
            pl.BlockSpec((BN, D), blk),
            pl.BlockSpec((BN, D), blk),
            pl.BlockSpec((D, D), full),
            pl.BlockSpec((D, D), full),
            pl.BlockSpec((1, D), full),
            pl.BlockSpec((D, D), full),
            pl.BlockSpec((1, D), full),
            pl.BlockSpec((D, D), full),
            pl.BlockSpec((1, D), full),
            pl.BlockSpec((D, 3), full),
            pl.BlockSpec((1, 3), full),
        ],
        out_specs=pl.BlockSpec((BN, 3), blk),
        out_shape=jax.ShapeDtypeStruct((N, 3), jnp.float32),
    )(h, a0, a1, a2, a3, w1a, w1b, b1, w2, b2, wd1, bd1, wd2, bd2)


def kernel(nodes, loc, edges, vel, edge_attr, _, W_emb, b_emb, We1, be1,
           We2, be2, Wn1, bn1, Wn2, bn2, Wd1, bd1, Wd2, bd2):
    row = edges[0]
    col = edges[1]
    rowp = [_pad_worker_idx(row[:E2]), _pad_worker_idx(row[E2:])]
    colp = [_pad_worker_idx(col[:E2]), _pad_worker_idx(col[E2:])]
    ea = [edge_attr[:E2], edge_attr[E2:]]
    zeros = jnp.zeros((NP, D), jnp.float32)

    def _wpq(i):
        A = We1[i, :D, :]
        B = We1[i, D:2 * D, :]
        return jnp.concatenate([A[:, 0::2], A[:, 1::2],
                                B[:, 0::2], B[:, 1::2]], axis=1)

    wpq = [_wpq(i) for i in range(L)]

    h, p, q = _embed(loc, vel, W_emb[:3], W_emb[3:], b_emb.reshape(1, D),
                     wpq[0])
    permi = jnp.concatenate([jnp.arange(0, D, 2), jnp.arange(1, D, 2)])
    for i in range(L):
        c2 = We1[i, 2 * D:, :][:, permi]
        b1 = be1[i][permi].reshape(1, D)
        w2 = We2[i][permi, :]
        b2 = be2[i].reshape(1, D)
        z0 = _gather_add(p, q, rowp[0], colp[0])
        m0 = _edge_mlp(z0, ea[0], c2, b1, w2, b2)
        z1 = _gather_add(p, q, rowp[1], colp[1])
        agg0 = _scatter_add(m0, rowp[0], zeros)
        m1 = _edge_mlp(z1, ea[1], c2, b1, w2, b2)
        agg1 = _scatter_add(m1, rowp[1], zeros)
        if i < L - 1:
            h, p, q = _node_update(h, agg0[:N], agg0[NP:NP + N],
                                   agg1[:N], agg1[NP:NP + N],
                                   Wn1[i, :D, :], Wn1[i, D:, :],
                                   bn1[i].reshape(1, D), Wn2[i],
                                   bn2[i].reshape(1, D), wpq[i + 1])
        else:
            return _node_decode(h, agg0[:N], agg0[NP:NP + N],
                                agg1[:N], agg1[NP:NP + N],
                                Wn1[i, :D, :], Wn1[i, D:, :],
                                bn1[i].reshape(1, D), Wn2[i],
                                bn2[i].reshape(1, D),
                                Wd1, bd1.reshape(1, D), Wd2,
                                bd2.reshape(1, 3))
